# Initial kernel scaffold; baseline (speedup 1.0000x reference)
import jax
import jax.numpy as jnp
from jax.experimental import pallas as pl


def kernel(x, edge_index, W1l, W1r, b1, gamma1, beta1, W2l, W2r, b2):
    raise NotImplementedError("write your pallas kernel here")


# speedup vs baseline: 3.3358x; 3.3358x over previous
import functools

import jax
import jax.numpy as jnp
from jax import lax
from jax.experimental import pallas as pl
from jax.experimental.pallas import tpu as pltpu
from jax.experimental.pallas import tpu_sc as plsc

N_NODES = 10000
N_EDGES = 320000
D = 128

NC = 2
NS = 16
CHUNK = 128
STEPS = 80
E_PAD = NC * NS * STEPS * CHUNK
N_ACC = 10240
ROWS_PER_TILE = N_ACC // NS


def _make_sc_agg(with_deg):
  mesh = plsc.VectorSubcoreMesh(core_axis_name="c", subcore_axis_name="s")
  out_type = jax.ShapeDtypeStruct((NC, N_ACC, D), jnp.float32)
  if with_deg:
    out_type = (out_type,
                jax.ShapeDtypeStruct((NC, NS, N_ACC), jnp.float32))
  scratch = [
      pltpu.VMEM((STEPS, CHUNK), jnp.int32),
      pltpu.VMEM((STEPS, CHUNK), jnp.int32),
      pltpu.VMEM((CHUNK, D), jnp.float32),
      pltpu.VMEM((16, D), jnp.float32),
      pltpu.VMEM_SHARED((N_ACC, D), jnp.float32),
  ]
  if with_deg:
    scratch.append(pltpu.VMEM((N_ACC,), jnp.float32))

  @functools.partial(
      pl.kernel, out_type=out_type, mesh=mesh, scratch_types=scratch,
      compiler_params=pltpu.CompilerParams(needs_layout_passes=False))
  def agg(table_hbm, src_hbm, dst_hbm, *rest):
    if with_deg:
      out_hbm, deg_hbm, src_v, dst_v, rows_v, zbuf, acc, hist = rest
    else:
      (out_hbm, src_v, dst_v, rows_v, zbuf, acc) = rest
    c = lax.axis_index("c")
    s = lax.axis_index("s")
    w = c * NS + s

    zeros16 = jnp.zeros((16,), jnp.float32)
    for r in range(16):
      for k in range(D // 16):
        zbuf[r, pl.ds(k * 16, 16)] = zeros16
    my_rows = s * ROWS_PER_TILE

    def zloop(i, carry):
      pltpu.sync_copy(zbuf, acc.at[pl.ds(my_rows + i * 16, 16)])
      return carry

    lax.fori_loop(0, ROWS_PER_TILE // 16, zloop, 0)

    if with_deg:
      def zhist(i, carry):
        hist[pl.ds(i * 16, 16)] = zeros16
        return carry

      lax.fori_loop(0, N_ACC // 16, zhist, 0)

    base = w * STEPS
    pltpu.sync_copy(src_hbm.at[pl.ds(base, STEPS)], src_v)
    pltpu.sync_copy(dst_hbm.at[pl.ds(base, STEPS)], dst_v)
    plsc.subcore_barrier()

    ones16 = jnp.ones((16,), jnp.float32)

    def body(j, carry):
      pltpu.sync_copy(table_hbm.at[src_v.at[j]], rows_v)
      pltpu.sync_copy(rows_v, acc.at[dst_v.at[j]], add=True)
      if with_deg:
        for k in range(CHUNK // 16):
          idx = dst_v[j, pl.ds(k * 16, 16)]
          plsc.addupdate_scatter(hist, [idx], ones16)
      return carry

    lax.fori_loop(0, STEPS, body, 0)
    plsc.subcore_barrier()

    pltpu.sync_copy(acc.at[pl.ds(my_rows, ROWS_PER_TILE)],
                    out_hbm.at[c, pl.ds(my_rows, ROWS_PER_TILE)])
    if with_deg:
      pltpu.sync_copy(hist, deg_hbm.at[c, s])

  return agg


_sc_agg_l1 = _make_sc_agg(True)
_sc_agg_l2 = _make_sc_agg(False)


def _tc_layer1(aggp, deg_t, x, wl, wr, b, gamma, beta):
  def body(aggp_ref, deg_ref, x_ref, wl_ref, wr_ref, b_ref, g_ref, be_ref,
           h_ref):
    agg = aggp_ref[0, :N_NODES, :] + aggp_ref[1, :N_NODES, :]
    deg = jnp.sum(deg_ref[...], axis=1, keepdims=True)[:N_NODES]
    inv = 1.0 / jnp.maximum(deg, 1.0)
    mean_agg = agg * inv
    h = (jnp.dot(mean_agg, wl_ref[...], preferred_element_type=jnp.float32)
         + b_ref[...][None, :]
         + jnp.dot(x_ref[...], wr_ref[...],
                   preferred_element_type=jnp.float32))
    mu = jnp.mean(h, axis=0)
    var = jnp.mean((h - mu[None, :]) ** 2, axis=0)
    hn = (h - mu[None, :]) / jnp.sqrt(var + 1e-5)
    hn = hn * g_ref[...][None, :] + be_ref[...][None, :]
    h_ref[...] = jnp.maximum(hn, 0.0)

  return pl.pallas_call(
      body,
      out_shape=jax.ShapeDtypeStruct((N_NODES, D), jnp.float32),
  )(aggp, deg_t, x, wl, wr, b, gamma, beta)


def _tc_layer2(aggp2, deg_t, h, wl, wr, b):
  def body(aggp2_ref, deg_ref, h_ref, wl_ref, wr_ref, b_ref, o_ref):
    agg = aggp2_ref[0, :N_NODES, :] + aggp2_ref[1, :N_NODES, :]
    deg = jnp.sum(deg_ref[...], axis=1, keepdims=True)[:N_NODES]
    inv = 1.0 / jnp.maximum(deg, 1.0)
    mean_agg = agg * inv
    o_ref[...] = (
        jnp.dot(mean_agg, wl_ref[...], preferred_element_type=jnp.float32)
        + b_ref[...][None, :]
        + jnp.dot(h_ref[...], wr_ref[...], preferred_element_type=jnp.float32))

  return pl.pallas_call(
      body,
      out_shape=jax.ShapeDtypeStruct((N_NODES, D), jnp.float32),
  )(aggp2, deg_t, h, wl, wr, b)


def kernel(x, edge_index, W1l, W1r, b1, gamma1, beta1, W2l, W2r, b2):
  src = edge_index[0].astype(jnp.int32)
  dst = edge_index[1].astype(jnp.int32)
  pad = E_PAD - N_EDGES
  src2d = jnp.concatenate(
      [src, jnp.zeros((pad,), jnp.int32)]).reshape(E_PAD // CHUNK, CHUNK)
  dst2d = jnp.concatenate(
      [dst, jnp.full((pad,), N_NODES, jnp.int32)]).reshape(
          E_PAD // CHUNK, CHUNK)

  aggp1, degp = _sc_agg_l1(x, src2d, dst2d)
  deg_t = degp.reshape(NC * NS, N_ACC).T
  h = _tc_layer1(aggp1, deg_t, x, W1l, W1r, b1, gamma1, beta1)
  aggp2 = _sc_agg_l2(h, src2d, dst2d)
  return _tc_layer2(aggp2, deg_t, h, W2l, W2r, b2)


# speedup vs baseline: 8.2996x; 2.4881x over previous
import functools

import jax
import jax.numpy as jnp
from jax import lax
from jax.experimental import pallas as pl
from jax.experimental.pallas import tpu as pltpu
from jax.experimental.pallas import tpu_sc as plsc

N_NODES = 10000
N_EDGES = 320000
D = 128

NC = 2
NS = 16
DH = D // NC
CHUNK = 128
STEPS = 160
PH = 20
NPHASE = STEPS // PH
E_PAD = NS * STEPS * CHUNK
N_ACC = 10240
ROWS_PER_TILE = N_ACC // NS


def _make_sc_agg(with_deg):
  mesh = plsc.VectorSubcoreMesh(core_axis_name="c", subcore_axis_name="s")
  out_type = jax.ShapeDtypeStruct((NC, N_ACC, DH), jnp.float32)
  if with_deg:
    out_type = (out_type, jax.ShapeDtypeStruct((NS, N_ACC), jnp.float32))
  scratch = [
      pltpu.VMEM((2, PH, CHUNK), jnp.int32),
      pltpu.VMEM((PH, CHUNK), jnp.int32),
      pltpu.VMEM((PH, CHUNK), jnp.int32),
      pltpu.VMEM((CHUNK, DH), jnp.float32),
      pltpu.VMEM((CHUNK, DH), jnp.float32),
      pltpu.VMEM((16, DH), jnp.float32),
      pltpu.VMEM_SHARED((N_NODES, DH), jnp.float32),
      pltpu.VMEM_SHARED((N_ACC, DH), jnp.float32),
      pltpu.SemaphoreType.DMA,
      pltpu.SemaphoreType.DMA,
      pltpu.SemaphoreType.DMA,
  ]
  if with_deg:
    scratch.append(pltpu.VMEM((N_ACC,), jnp.float32))

  @functools.partial(
      pl.kernel, out_type=out_type, mesh=mesh, scratch_types=scratch,
      compiler_params=pltpu.CompilerParams(needs_layout_passes=False,
                                           use_tc_tiling_on_sc=False))
  def agg(table_hbm, edges_hbm, *rest):
    if with_deg:
      (out_hbm, deg_hbm, pbuf, src_v, dst_v, rows0, rows1, zbuf, table, acc,
       gsem0, gsem1, psem, hist) = rest
    else:
      (out_hbm, pbuf, src_v, dst_v, rows0, rows1, zbuf, table, acc,
       gsem0, gsem1, psem) = rest
    c = lax.axis_index("c")
    s = lax.axis_index("s")

    zeros16 = jnp.zeros((16,), jnp.float32)
    for r in range(16):
      for k in range(DH // 16):
        zbuf[r, pl.ds(k * 16, 16)] = zeros16
    my_rows = s * ROWS_PER_TILE

    def zloop(i, carry):
      pltpu.sync_copy(zbuf, acc.at[pl.ds(my_rows + i * 16, 16)])
      return carry

    lax.fori_loop(0, ROWS_PER_TILE // 16, zloop, 0)

    tstart = jnp.minimum(s * 640, N_NODES - 640)
    pltpu.sync_copy(table_hbm.at[c, pl.ds(tstart, 640)],
                    table.at[pl.ds(tstart, 640)])

    if with_deg:
      @pl.when(c == 0)
      def _():
        def zhist(i, carry):
          hist[pl.ds(i * 16, 16)] = zeros16
          return carry

        lax.fori_loop(0, N_ACC // 16, zhist, 0)

    base = s * STEPS
    pltpu.sync_copy(edges_hbm.at[pl.ds(base, PH)], pbuf.at[0])
    plsc.subcore_barrier()

    ones16 = jnp.ones((16,), jnp.float32)

    def gstart(j, rows, sem):
      pltpu.async_copy(table.at[src_v.at[j]], rows, sem)

    def gwait(rows, sem):
      pltpu.make_async_copy(table.at[src_v.at[0]], rows, sem).wait()

    def hist_add(j):
      if with_deg:
        @pl.when(c == 0)
        def _():
          for k in range(CHUNK // 16):
            idx = dst_v[j, pl.ds(k * 16, 16)]
            plsc.addupdate_scatter(hist, [idx], ones16)

    def phase_body(p, carry):
      pb = lax.rem(p, 2)
      @pl.when(p + 1 < NPHASE)
      def _():
        pltpu.async_copy(edges_hbm.at[pl.ds(base + (p + 1) * PH, PH)],
                         pbuf.at[lax.rem(p + 1, 2)], psem)

      def unpack(r, carry2):
        for k in range(CHUNK // 16):
          q = pbuf[pb, r, pl.ds(k * 16, 16)]
          src_v[r, pl.ds(k * 16, 16)] = q & 0xFFFF
          dst_v[r, pl.ds(k * 16, 16)] = lax.shift_right_logical(q, 16)
        return carry2

      lax.fori_loop(0, PH, unpack, 0)

      gstart(0, rows0, gsem0)

      def body(g, carry2):
        j0 = 2 * g
        gwait(rows0, gsem0)
        gstart(j0 + 1, rows1, gsem1)
        pltpu.sync_copy(rows0, acc.at[dst_v.at[j0]], add=True)
        hist_add(j0)
        gwait(rows1, gsem1)
        gstart(jnp.minimum(j0 + 2, PH - 1), rows0, gsem0)
        pltpu.sync_copy(rows1, acc.at[dst_v.at[j0 + 1]], add=True)
        hist_add(j0 + 1)
        return carry2

      lax.fori_loop(0, PH // 2, body, 0)
      gwait(rows0, gsem0)

      @pl.when(p + 1 < NPHASE)
      def _():
        pltpu.make_async_copy(edges_hbm.at[pl.ds(base, PH)], pbuf.at[0],
                              psem).wait()

      return carry

    lax.fori_loop(0, NPHASE, phase_body, 0)
    plsc.subcore_barrier()

    pltpu.sync_copy(acc.at[pl.ds(my_rows, ROWS_PER_TILE)],
                    out_hbm.at[c, pl.ds(my_rows, ROWS_PER_TILE)])
    if with_deg:
      @pl.when(c == 0)
      def _():
        pltpu.sync_copy(hist, deg_hbm.at[s])

  return agg


_sc_agg_l1 = _make_sc_agg(True)
_sc_agg_l2 = _make_sc_agg(False)


def _tc_layer1(aggp, deg_t, x, wl, wr, b, gamma, beta):
  def body(aggp_ref, deg_ref, x_ref, wl_ref, wr_ref, b_ref, g_ref, be_ref,
           h_ref):
    agg = jnp.concatenate(
        [aggp_ref[0, :N_NODES, :], aggp_ref[1, :N_NODES, :]], axis=1)
    deg = jnp.sum(deg_ref[...], axis=1, keepdims=True)[:N_NODES]
    inv = 1.0 / jnp.maximum(deg, 1.0)
    mean_agg = agg * inv
    h = (jnp.dot(mean_agg, wl_ref[...], preferred_element_type=jnp.float32)
         + b_ref[...][None, :]
         + jnp.dot(x_ref[...], wr_ref[...],
                   preferred_element_type=jnp.float32))
    mu = jnp.mean(h, axis=0)
    var = jnp.mean((h - mu[None, :]) ** 2, axis=0)
    hn = (h - mu[None, :]) / jnp.sqrt(var + 1e-5)
    hn = hn * g_ref[...][None, :] + be_ref[...][None, :]
    h = jnp.maximum(hn, 0.0)
    h_ref[...] = h

  return pl.pallas_call(
      body,
      out_shape=jax.ShapeDtypeStruct((N_NODES, D), jnp.float32),
  )(aggp, deg_t, x, wl, wr, b, gamma, beta)


def _tc_layer2(aggp2, deg_t, h, wl, wr, b):
  def body(aggp2_ref, deg_ref, h_ref, wl_ref, wr_ref, b_ref, o_ref):
    agg = jnp.concatenate(
        [aggp2_ref[0, :N_NODES, :], aggp2_ref[1, :N_NODES, :]], axis=1)
    deg = jnp.sum(deg_ref[...], axis=1, keepdims=True)[:N_NODES]
    inv = 1.0 / jnp.maximum(deg, 1.0)
    mean_agg = agg * inv
    o_ref[...] = (
        jnp.dot(mean_agg, wl_ref[...], preferred_element_type=jnp.float32)
        + b_ref[...][None, :]
        + jnp.dot(h_ref[...], wr_ref[...], preferred_element_type=jnp.float32))

  return pl.pallas_call(
      body,
      out_shape=jax.ShapeDtypeStruct((N_NODES, D), jnp.float32),
  )(aggp2, deg_t, h, wl, wr, b)


def kernel(x, edge_index, W1l, W1r, b1, gamma1, beta1, W2l, W2r, b2):
  src = edge_index[0].astype(jnp.int32)
  dst = edge_index[1].astype(jnp.int32)
  pad = E_PAD - N_EDGES
  packed = jnp.concatenate(
      [(dst << 16) | src,
       jnp.full((pad,), N_NODES << 16, jnp.int32)]).reshape(
           E_PAD // CHUNK, CHUNK)

  xh = jnp.stack([x[:, :DH], x[:, DH:]])
  aggp1, degp = _sc_agg_l1(xh, packed)
  deg_t = degp.reshape(NS, N_ACC).T
  h = _tc_layer1(aggp1, deg_t, x, W1l, W1r, b1, gamma1, beta1)
  hh = jnp.stack([h[:, :DH], h[:, DH:]])
  aggp2 = _sc_agg_l2(hh, packed)
  return _tc_layer2(aggp2, deg_t, h, W2l, W2r, b2)


# speedup vs baseline: 9.6656x; 1.1646x over previous
import functools

import jax
import jax.numpy as jnp
from jax import lax
from jax.experimental import pallas as pl
from jax.experimental.pallas import tpu as pltpu
from jax.experimental.pallas import tpu_sc as plsc

N_NODES = 10000
N_EDGES = 320000
D = 128

NC = 2
NS = 16
DH = D // NC
CHUNK = 128
STEPS = 160
PH = 20
NPHASE = STEPS // PH
E_PAD = NS * STEPS * CHUNK
N_ACC = 10240
ROWS_PER_TILE = N_ACC // NS


def _make_sc_agg(with_deg):
  mesh = plsc.VectorSubcoreMesh(core_axis_name="c", subcore_axis_name="s")
  out_type = jax.ShapeDtypeStruct((NC, N_ACC, DH), jnp.float32)
  if with_deg:
    out_type = (out_type, jax.ShapeDtypeStruct((NS, N_ACC), jnp.float32))
  scratch = [
      pltpu.VMEM((2, PH, CHUNK), jnp.int32),
      pltpu.VMEM((2, PH, CHUNK), jnp.int32),
      pltpu.VMEM((CHUNK, DH), jnp.float32),
      pltpu.VMEM((CHUNK, DH), jnp.float32),
      pltpu.VMEM((16, DH), jnp.float32),
      pltpu.VMEM_SHARED((N_NODES, DH), jnp.float32),
      pltpu.VMEM_SHARED((N_ACC, DH), jnp.float32),
      pltpu.SemaphoreType.DMA,
      pltpu.SemaphoreType.DMA,
      pltpu.SemaphoreType.DMA,
  ]
  if with_deg:
    scratch.append(pltpu.VMEM((N_ACC,), jnp.float32))

  @functools.partial(
      pl.kernel, out_type=out_type, mesh=mesh, scratch_types=scratch,
      compiler_params=pltpu.CompilerParams(needs_layout_passes=False,
                                           use_tc_tiling_on_sc=False))
  def agg(table_hbm, src_hbm, dst_hbm, *rest):
    if with_deg:
      (out_hbm, deg_hbm, src_v, dst_v, rows0, rows1, zbuf, table, acc,
       gsem0, gsem1, psem, hist) = rest
    else:
      (out_hbm, src_v, dst_v, rows0, rows1, zbuf, table, acc,
       gsem0, gsem1, psem) = rest
    c = lax.axis_index("c")
    s = lax.axis_index("s")

    zeros16 = jnp.zeros((16,), jnp.float32)
    for r in range(16):
      for k in range(DH // 16):
        zbuf[r, pl.ds(k * 16, 16)] = zeros16
    my_rows = s * ROWS_PER_TILE

    def zloop(i, carry):
      pltpu.sync_copy(zbuf, acc.at[pl.ds(my_rows + i * 16, 16)])
      return carry

    lax.fori_loop(0, ROWS_PER_TILE // 16, zloop, 0)

    tstart = jnp.minimum(s * 640, N_NODES - 640)
    pltpu.sync_copy(table_hbm.at[pl.ds(tstart, 640), pl.ds(c * DH, DH)],
                    table.at[pl.ds(tstart, 640)])

    if with_deg:
      @pl.when(c == 0)
      def _():
        def zhist(i, carry):
          hist[pl.ds(i * 16, 16)] = zeros16
          return carry

        lax.fori_loop(0, N_ACC // 16, zhist, 0)

    base = s * STEPS
    pltpu.sync_copy(src_hbm.at[pl.ds(base, PH)], src_v.at[0])
    pltpu.sync_copy(dst_hbm.at[pl.ds(base, PH)], dst_v.at[0])
    plsc.subcore_barrier()

    ones16 = jnp.ones((16,), jnp.float32)

    def gstart(pb, j, rows, sem):
      pltpu.async_copy(table.at[src_v.at[pb, j]], rows, sem)

    def gwait(rows, sem):
      pltpu.make_async_copy(table.at[src_v.at[0, 0]], rows, sem).wait()

    def hist_add(pb, j):
      if with_deg:
        @pl.when(c == 0)
        def _():
          for k in range(CHUNK // 16):
            idx = dst_v[pb, j, pl.ds(k * 16, 16)]
            plsc.addupdate_scatter(hist, [idx], ones16)

    def phase_body(p, carry):
      pb = lax.rem(p, 2)
      pn = lax.rem(p + 1, 2)
      @pl.when(p + 1 < NPHASE)
      def _():
        nxt = base + (p + 1) * PH
        pltpu.async_copy(src_hbm.at[pl.ds(nxt, PH)], src_v.at[pn], psem)
        pltpu.async_copy(dst_hbm.at[pl.ds(nxt, PH)], dst_v.at[pn], psem)

      gstart(pb, 0, rows0, gsem0)

      def body(g, carry2):
        j0 = 2 * g
        gwait(rows0, gsem0)
        gstart(pb, j0 + 1, rows1, gsem1)
        pltpu.sync_copy(rows0, acc.at[dst_v.at[pb, j0]], add=True)
        hist_add(pb, j0)
        gwait(rows1, gsem1)
        gstart(pb, jnp.minimum(j0 + 2, PH - 1), rows0, gsem0)
        pltpu.sync_copy(rows1, acc.at[dst_v.at[pb, j0 + 1]], add=True)
        hist_add(pb, j0 + 1)
        return carry2

      lax.fori_loop(0, PH // 2, body, 0)
      gwait(rows0, gsem0)

      @pl.when(p + 1 < NPHASE)
      def _():
        pltpu.make_async_copy(src_hbm.at[pl.ds(base, PH)], src_v.at[0],
                              psem).wait()
        pltpu.make_async_copy(dst_hbm.at[pl.ds(base, PH)], dst_v.at[0],
                              psem).wait()

      return carry

    lax.fori_loop(0, NPHASE, phase_body, 0)
    plsc.subcore_barrier()

    pltpu.sync_copy(acc.at[pl.ds(my_rows, ROWS_PER_TILE)],
                    out_hbm.at[c, pl.ds(my_rows, ROWS_PER_TILE)])
    if with_deg:
      @pl.when(c == 0)
      def _():
        pltpu.sync_copy(hist, deg_hbm.at[s])

  return agg


_sc_agg_l1 = _make_sc_agg(True)
_sc_agg_l2 = _make_sc_agg(False)


def _mean_agg(aggp_ref, deg_ref, wl_ref):
  deg16 = deg_ref[...]
  deg_cols = jnp.transpose(deg16)[:N_NODES]
  deg = jnp.sum(deg_cols, axis=1, keepdims=True)
  inv = 1.0 / jnp.maximum(deg, 1.0)
  left = jnp.dot(aggp_ref[0, :N_NODES, :] * inv, wl_ref[:DH, :],
                 preferred_element_type=jnp.float32)
  right = jnp.dot(aggp_ref[1, :N_NODES, :] * inv, wl_ref[DH:, :],
                  preferred_element_type=jnp.float32)
  return left + right


def _tc_layer1(aggp, degp, x, wl, wr, b, gamma, beta):
  def body(aggp_ref, deg_ref, x_ref, wl_ref, wr_ref, b_ref, g_ref, be_ref,
           h_ref):
    h = (_mean_agg(aggp_ref, deg_ref, wl_ref)
         + b_ref[...][None, :]
         + jnp.dot(x_ref[...], wr_ref[...],
                   preferred_element_type=jnp.float32))
    mu = jnp.mean(h, axis=0)
    var = jnp.mean((h - mu[None, :]) ** 2, axis=0)
    hn = (h - mu[None, :]) / jnp.sqrt(var + 1e-5)
    hn = hn * g_ref[...][None, :] + be_ref[...][None, :]
    h_ref[...] = jnp.maximum(hn, 0.0)

  return pl.pallas_call(
      body,
      out_shape=jax.ShapeDtypeStruct((N_NODES, D), jnp.float32),
  )(aggp, degp, x, wl, wr, b, gamma, beta)


def _tc_layer2(aggp2, degp, h, wl, wr, b):
  def body(aggp2_ref, deg_ref, h_ref, wl_ref, wr_ref, b_ref, o_ref):
    o_ref[...] = (
        _mean_agg(aggp2_ref, deg_ref, wl_ref)
        + b_ref[...][None, :]
        + jnp.dot(h_ref[...], wr_ref[...], preferred_element_type=jnp.float32))

  return pl.pallas_call(
      body,
      out_shape=jax.ShapeDtypeStruct((N_NODES, D), jnp.float32),
  )(aggp2, degp, h, wl, wr, b)


def kernel(x, edge_index, W1l, W1r, b1, gamma1, beta1, W2l, W2r, b2):
  src = edge_index[0].astype(jnp.int32)
  dst = edge_index[1].astype(jnp.int32)
  pad = E_PAD - N_EDGES
  src2d = jnp.concatenate(
      [src, jnp.zeros((pad,), jnp.int32)]).reshape(E_PAD // CHUNK, CHUNK)
  dst2d = jnp.concatenate(
      [dst, jnp.full((pad,), N_NODES, jnp.int32)]).reshape(
          E_PAD // CHUNK, CHUNK)

  aggp1, degp = _sc_agg_l1(x, src2d, dst2d)
  h = _tc_layer1(aggp1, degp, x, W1l, W1r, b1, gamma1, beta1)
  aggp2 = _sc_agg_l2(h, src2d, dst2d)
  return _tc_layer2(aggp2, degp, h, W2l, W2r, b2)


# speedup vs baseline: 9.6796x; 1.0014x over previous
import functools

import jax
import jax.numpy as jnp
from jax import lax
from jax.experimental import pallas as pl
from jax.experimental.pallas import tpu as pltpu
from jax.experimental.pallas import tpu_sc as plsc

N_NODES = 10000
N_EDGES = 320000
D = 128

NC = 2
NS = 16
DH = D // NC
CHUNK = 128
STEPS = 160
PH = 20
NPHASE = STEPS // PH
E_PAD = NS * STEPS * CHUNK
N_ACC = 10240
ROWS_PER_TILE = N_ACC // NS


def _make_sc_agg(with_deg):
  mesh = plsc.VectorSubcoreMesh(core_axis_name="c", subcore_axis_name="s")
  out_type = jax.ShapeDtypeStruct((NC, N_ACC, DH), jnp.float32)
  if with_deg:
    out_type = (out_type, jax.ShapeDtypeStruct((NS, N_ACC), jnp.float32))
  scratch = [
      pltpu.VMEM((2, PH, CHUNK), jnp.int32),
      pltpu.VMEM((2, PH, CHUNK), jnp.int32),
      pltpu.VMEM((3, CHUNK, DH), jnp.float32),
      pltpu.VMEM((16, DH), jnp.float32),
      pltpu.VMEM_SHARED((N_NODES, DH), jnp.float32),
      pltpu.VMEM_SHARED((N_ACC, DH), jnp.float32),
      pltpu.SemaphoreType.DMA((2,)),
      pltpu.SemaphoreType.DMA,
      pltpu.SemaphoreType.DMA,
  ]
  if with_deg:
    scratch.append(pltpu.VMEM((N_ACC,), jnp.float32))

  @functools.partial(
      pl.kernel, out_type=out_type, mesh=mesh, scratch_types=scratch,
      compiler_params=pltpu.CompilerParams(needs_layout_passes=False,
                                           use_tc_tiling_on_sc=False))
  def agg(table_hbm, src_hbm, dst_hbm, *rest):
    if with_deg:
      (out_hbm, deg_hbm, src_v, dst_v, rows, zbuf, table, acc,
       gsem, ssem, psem, hist) = rest
    else:
      (out_hbm, src_v, dst_v, rows, zbuf, table, acc,
       gsem, ssem, psem) = rest
    c = lax.axis_index("c")
    s = lax.axis_index("s")

    zeros16 = jnp.zeros((16,), jnp.float32)
    for r in range(16):
      for k in range(DH // 16):
        zbuf[r, pl.ds(k * 16, 16)] = zeros16
    my_rows = s * ROWS_PER_TILE

    def zloop(i, carry):
      pltpu.sync_copy(zbuf, acc.at[pl.ds(my_rows + i * 16, 16)])
      return carry

    lax.fori_loop(0, ROWS_PER_TILE // 16, zloop, 0)

    tstart = jnp.minimum(s * 640, N_NODES - 640)
    pltpu.sync_copy(table_hbm.at[pl.ds(tstart, 640), pl.ds(c * DH, DH)],
                    table.at[pl.ds(tstart, 640)])

    if with_deg:
      @pl.when(c == 0)
      def _():
        def zhist(i, carry):
          hist[pl.ds(i * 16, 16)] = zeros16
          return carry

        lax.fori_loop(0, N_ACC // 16, zhist, 0)

    base = s * STEPS
    pltpu.sync_copy(src_hbm.at[pl.ds(base, PH)], src_v.at[0])
    pltpu.sync_copy(dst_hbm.at[pl.ds(base, PH)], dst_v.at[0])
    plsc.subcore_barrier()

    ones16 = jnp.ones((16,), jnp.float32)

    def gstart(pb, j, buf):
      pltpu.async_copy(table.at[src_v.at[pb, j]], rows.at[buf],
                       gsem.at[lax.rem(j, 2)])

    def gwait(j):
      pltpu.make_async_copy(table.at[src_v.at[0, 0]], rows.at[0],
                            gsem.at[lax.rem(j, 2)]).wait()

    def swait():
      pltpu.make_async_copy(rows.at[0], acc.at[dst_v.at[0, 0]], ssem).wait()

    def hist_add(pb, j):
      if with_deg:
        @pl.when(c == 0)
        def _():
          for k in range(CHUNK // 16):
            idx = dst_v[pb, j, pl.ds(k * 16, 16)]
            plsc.addupdate_scatter(hist, [idx], ones16)

    def phase_body(p, carry):
      pb = lax.rem(p, 2)
      pn = lax.rem(p + 1, 2)
      @pl.when(p + 1 < NPHASE)
      def _():
        nxt = base + (p + 1) * PH
        pltpu.async_copy(src_hbm.at[pl.ds(nxt, PH)], src_v.at[pn], psem)
        pltpu.async_copy(dst_hbm.at[pl.ds(nxt, PH)], dst_v.at[pn], psem)

      gstart(pb, 0, 0)
      gstart(pb, 1, 1)

      def body(j, carry2):
        bj = lax.rem(j, 3)
        @pl.when(j >= 1)
        def _():
          swait()

        gwait(j)
        pltpu.async_copy(rows.at[bj], acc.at[dst_v.at[pb, j]], ssem,
                         add=True)
        hist_add(pb, j)

        @pl.when(j + 2 < PH)
        def _():
          gstart(pb, j + 2, lax.rem(j + 2, 3))

        return carry2

      lax.fori_loop(0, PH, body, 0)
      swait()

      @pl.when(p + 1 < NPHASE)
      def _():
        pltpu.make_async_copy(src_hbm.at[pl.ds(base, PH)], src_v.at[0],
                              psem).wait()
        pltpu.make_async_copy(dst_hbm.at[pl.ds(base, PH)], dst_v.at[0],
                              psem).wait()

      return carry

    lax.fori_loop(0, NPHASE, phase_body, 0)
    plsc.subcore_barrier()

    pltpu.sync_copy(acc.at[pl.ds(my_rows, ROWS_PER_TILE)],
                    out_hbm.at[c, pl.ds(my_rows, ROWS_PER_TILE)])
    if with_deg:
      @pl.when(c == 0)
      def _():
        pltpu.sync_copy(hist, deg_hbm.at[s])

  return agg


_sc_agg_l1 = _make_sc_agg(True)
_sc_agg_l2 = _make_sc_agg(False)


def _mean_agg(aggp_ref, deg_ref, wl_ref):
  deg16 = deg_ref[...]
  deg_cols = jnp.transpose(deg16)[:N_NODES]
  deg = jnp.sum(deg_cols, axis=1, keepdims=True)
  inv = 1.0 / jnp.maximum(deg, 1.0)
  left = jnp.dot(aggp_ref[0, :N_NODES, :] * inv, wl_ref[:DH, :],
                 preferred_element_type=jnp.float32)
  right = jnp.dot(aggp_ref[1, :N_NODES, :] * inv, wl_ref[DH:, :],
                  preferred_element_type=jnp.float32)
  return left + right


def _tc_layer1(aggp, degp, x, wl, wr, b, gamma, beta):
  def body(aggp_ref, deg_ref, x_ref, wl_ref, wr_ref, b_ref, g_ref, be_ref,
           h_ref):
    h = (_mean_agg(aggp_ref, deg_ref, wl_ref)
         + b_ref[...][None, :]
         + jnp.dot(x_ref[...], wr_ref[...],
                   preferred_element_type=jnp.float32))
    mu = jnp.mean(h, axis=0)
    var = jnp.mean((h - mu[None, :]) ** 2, axis=0)
    hn = (h - mu[None, :]) / jnp.sqrt(var + 1e-5)
    hn = hn * g_ref[...][None, :] + be_ref[...][None, :]
    h_ref[...] = jnp.maximum(hn, 0.0)

  return pl.pallas_call(
      body,
      out_shape=jax.ShapeDtypeStruct((N_NODES, D), jnp.float32),
  )(aggp, degp, x, wl, wr, b, gamma, beta)


def _tc_layer2(aggp2, degp, h, wl, wr, b):
  def body(aggp2_ref, deg_ref, h_ref, wl_ref, wr_ref, b_ref, o_ref):
    o_ref[...] = (
        _mean_agg(aggp2_ref, deg_ref, wl_ref)
        + b_ref[...][None, :]
        + jnp.dot(h_ref[...], wr_ref[...], preferred_element_type=jnp.float32))

  return pl.pallas_call(
      body,
      out_shape=jax.ShapeDtypeStruct((N_NODES, D), jnp.float32),
  )(aggp2, degp, h, wl, wr, b)


def kernel(x, edge_index, W1l, W1r, b1, gamma1, beta1, W2l, W2r, b2):
  src = edge_index[0].astype(jnp.int32)
  dst = edge_index[1].astype(jnp.int32)
  pad = E_PAD - N_EDGES
  src2d = jnp.concatenate(
      [src, jnp.zeros((pad,), jnp.int32)]).reshape(E_PAD // CHUNK, CHUNK)
  dst2d = jnp.concatenate(
      [dst, jnp.full((pad,), N_NODES, jnp.int32)]).reshape(
          E_PAD // CHUNK, CHUNK)

  aggp1, degp = _sc_agg_l1(x, src2d, dst2d)
  h = _tc_layer1(aggp1, degp, x, W1l, W1r, b1, gamma1, beta1)
  aggp2 = _sc_agg_l2(h, src2d, dst2d)
  return _tc_layer2(aggp2, degp, h, W2l, W2r, b2)


# speedup vs baseline: 10.1108x; 1.0445x over previous
import functools

import jax
import jax.numpy as jnp
from jax import lax
from jax.experimental import pallas as pl
from jax.experimental.pallas import tpu as pltpu
from jax.experimental.pallas import tpu_sc as plsc

N_NODES = 10000
N_EDGES = 320000
D = 128

NC = 2
NS = 16
DH = D // NC
CHUNK = 128
NROWS = N_EDGES // CHUNK
STEPS = 156
PH = 12
NPHASE = STEPS // PH
NTAIL = NROWS - NS * STEPS
N_ACC = 10240
ROWS_PER_TILE = N_ACC // NS


def _make_sc_agg(with_deg):
  mesh = plsc.VectorSubcoreMesh(core_axis_name="c", subcore_axis_name="s")
  out_type = jax.ShapeDtypeStruct((NC, N_ACC, DH), jnp.float32)
  if with_deg:
    out_type = (out_type, jax.ShapeDtypeStruct((NS, N_ACC), jnp.float32))
  scratch = [
      pltpu.VMEM((2, PH, CHUNK), jnp.int32),
      pltpu.VMEM((2, PH, CHUNK), jnp.int32),
      pltpu.VMEM((3, CHUNK, DH), jnp.float32),
      pltpu.VMEM((16, DH), jnp.float32),
      pltpu.VMEM_SHARED((N_NODES, DH), jnp.float32),
      pltpu.VMEM_SHARED((N_ACC, DH), jnp.float32),
      pltpu.SemaphoreType.DMA((2,)),
      pltpu.SemaphoreType.DMA,
      pltpu.SemaphoreType.DMA,
  ]
  if with_deg:
    scratch.append(pltpu.VMEM((N_ACC,), jnp.float32))

  @functools.partial(
      pl.kernel, out_type=out_type, mesh=mesh, scratch_types=scratch,
      compiler_params=pltpu.CompilerParams(needs_layout_passes=False,
                                           use_tc_tiling_on_sc=False))
  def agg(table_hbm, edges_hbm, *rest):
    if with_deg:
      (out_hbm, deg_hbm, src_v, dst_v, rows, zbuf, table, acc,
       gsem, ssem, psem, hist) = rest
    else:
      (out_hbm, src_v, dst_v, rows, zbuf, table, acc,
       gsem, ssem, psem) = rest
    c = lax.axis_index("c")
    s = lax.axis_index("s")

    zeros16 = jnp.zeros((16,), jnp.float32)
    for r in range(16):
      for k in range(DH // 16):
        zbuf[r, pl.ds(k * 16, 16)] = zeros16
    my_rows = s * ROWS_PER_TILE

    def zloop(i, carry):
      pltpu.sync_copy(zbuf, acc.at[pl.ds(my_rows + i * 16, 16)])
      return carry

    lax.fori_loop(0, ROWS_PER_TILE // 16, zloop, 0)

    tstart = jnp.minimum(s * 640, N_NODES - 640)
    pltpu.sync_copy(table_hbm.at[pl.ds(tstart, 640), pl.ds(c * DH, DH)],
                    table.at[pl.ds(tstart, 640)])

    if with_deg:
      @pl.when(c == 0)
      def _():
        def zhist(i, carry):
          hist[pl.ds(i * 16, 16)] = zeros16
          return carry

        lax.fori_loop(0, N_ACC // 16, zhist, 0)

    base = s * STEPS
    pltpu.sync_copy(edges_hbm.at[0, pl.ds(base, PH)], src_v.at[0])
    pltpu.sync_copy(edges_hbm.at[1, pl.ds(base, PH)], dst_v.at[0])
    plsc.subcore_barrier()

    ones16 = jnp.ones((16,), jnp.float32)

    def gstart(pb, j, buf):
      pltpu.async_copy(table.at[src_v.at[pb, j]], rows.at[buf],
                       gsem.at[lax.rem(j, 2)])

    def gwait(j):
      pltpu.make_async_copy(table.at[src_v.at[0, 0]], rows.at[0],
                            gsem.at[lax.rem(j, 2)]).wait()

    def swait():
      pltpu.make_async_copy(rows.at[0], acc.at[dst_v.at[0, 0]], ssem).wait()

    def hist_add(pb, j):
      if with_deg:
        @pl.when(c == 0)
        def _():
          for k in range(CHUNK // 16):
            idx = dst_v[pb, j, pl.ds(k * 16, 16)]
            plsc.addupdate_scatter(hist, [idx], ones16)

    def phase_body(p, carry):
      pb = lax.rem(p, 2)
      pn = lax.rem(p + 1, 2)
      @pl.when(p + 1 < NPHASE)
      def _():
        nxt = base + (p + 1) * PH
        pltpu.async_copy(edges_hbm.at[0, pl.ds(nxt, PH)], src_v.at[pn], psem)
        pltpu.async_copy(edges_hbm.at[1, pl.ds(nxt, PH)], dst_v.at[pn], psem)

      gstart(pb, 0, 0)
      gstart(pb, 1, 1)

      def body(j, carry2):
        bj = lax.rem(j, 3)
        @pl.when(j >= 1)
        def _():
          swait()

        gwait(j)
        pltpu.async_copy(rows.at[bj], acc.at[dst_v.at[pb, j]], ssem,
                         add=True)
        hist_add(pb, j)

        @pl.when(j + 2 < PH)
        def _():
          gstart(pb, j + 2, lax.rem(j + 2, 3))

        return carry2

      lax.fori_loop(0, PH, body, 0)
      swait()

      @pl.when(p + 1 < NPHASE)
      def _():
        pltpu.make_async_copy(edges_hbm.at[0, pl.ds(base, PH)], src_v.at[0],
                              psem).wait()
        pltpu.make_async_copy(edges_hbm.at[1, pl.ds(base, PH)], dst_v.at[0],
                              psem).wait()

      return carry

    lax.fori_loop(0, NPHASE, phase_body, 0)

    @pl.when(s < NTAIL)
    def _():
      trow = NS * STEPS + s
      pltpu.sync_copy(edges_hbm.at[0, pl.ds(trow, 1)],
                      src_v.at[0, pl.ds(0, 1)])
      pltpu.sync_copy(edges_hbm.at[1, pl.ds(trow, 1)],
                      dst_v.at[0, pl.ds(0, 1)])
      pltpu.sync_copy(table.at[src_v.at[0, 0]], rows.at[0])
      pltpu.sync_copy(rows.at[0], acc.at[dst_v.at[0, 0]], add=True)
      hist_add(0, 0)

    plsc.subcore_barrier()

    pltpu.sync_copy(acc.at[pl.ds(my_rows, ROWS_PER_TILE)],
                    out_hbm.at[c, pl.ds(my_rows, ROWS_PER_TILE)])
    if with_deg:
      @pl.when(c == 0)
      def _():
        pltpu.sync_copy(hist, deg_hbm.at[s])

  return agg


_sc_agg_l1 = _make_sc_agg(True)
_sc_agg_l2 = _make_sc_agg(False)

NP2 = N_NODES // 2


def _mean_agg_matmul(aggp_ref, deg_ref, wl_ref):
  deg_cols = jnp.transpose(deg_ref[...])[:N_NODES]
  deg = jnp.sum(deg_cols, axis=1, keepdims=True)
  inv2 = (1.0 / jnp.maximum(deg, 1.0)).reshape(NP2, 2)
  scale = jnp.concatenate(
      [jnp.broadcast_to(inv2[:, 0:1], (NP2, DH)),
       jnp.broadcast_to(inv2[:, 1:2], (NP2, DH))], axis=1)
  zz = jnp.zeros((DH, D), jnp.float32)
  packed = None
  for cc in range(NC):
    wl_c = wl_ref[pl.ds(cc * DH, DH), :]
    wbig = jnp.concatenate(
        [jnp.concatenate([wl_c, zz], axis=1),
         jnp.concatenate([zz, wl_c], axis=1)], axis=0)
    term = jnp.dot(aggp_ref[cc, :NP2, :] * scale, wbig,
                   preferred_element_type=jnp.float32)
    packed = term if packed is None else packed + term
  return packed.reshape(N_NODES, D)


def _tc_layer1(aggp, degp, x, wl, wr, b, gamma, beta):
  def body(aggp_ref, deg_ref, x_ref, wl_ref, wr_ref, b_ref, g_ref, be_ref,
           h_ref):
    h = (_mean_agg_matmul(aggp_ref, deg_ref, wl_ref)
         + b_ref[...][None, :]
         + jnp.dot(x_ref[...], wr_ref[...],
                   preferred_element_type=jnp.float32))
    mu = jnp.mean(h, axis=0)
    var = jnp.mean((h - mu[None, :]) ** 2, axis=0)
    hn = (h - mu[None, :]) / jnp.sqrt(var + 1e-5)
    hn = hn * g_ref[...][None, :] + be_ref[...][None, :]
    h_ref[...] = jnp.maximum(hn, 0.0)

  return pl.pallas_call(
      body,
      out_shape=jax.ShapeDtypeStruct((N_NODES, D), jnp.float32),
  )(aggp, degp, x, wl, wr, b, gamma, beta)


def _tc_layer2(aggp2, degp, h, wl, wr, b):
  def body(aggp2_ref, deg_ref, h_ref, wl_ref, wr_ref, b_ref, o_ref):
    o_ref[...] = (
        _mean_agg_matmul(aggp2_ref, deg_ref, wl_ref)
        + b_ref[...][None, :]
        + jnp.dot(h_ref[...], wr_ref[...], preferred_element_type=jnp.float32))

  return pl.pallas_call(
      body,
      out_shape=jax.ShapeDtypeStruct((N_NODES, D), jnp.float32),
  )(aggp2, degp, h, wl, wr, b)


def kernel(x, edge_index, W1l, W1r, b1, gamma1, beta1, W2l, W2r, b2):
  edges = edge_index.astype(jnp.int32).reshape(2, NROWS, CHUNK)

  aggp1, degp = _sc_agg_l1(x, edges)
  aggp1 = aggp1.reshape(NC, N_ACC // 2, 2 * DH)
  h = _tc_layer1(aggp1, degp, x, W1l, W1r, b1, gamma1, beta1)
  aggp2 = _sc_agg_l2(h, edges).reshape(NC, N_ACC // 2, 2 * DH)
  return _tc_layer2(aggp2, degp, h, W2l, W2r, b2)


# speedup vs baseline: 10.5371x; 1.0422x over previous
import functools

import jax
import jax.numpy as jnp
from jax import lax
from jax.experimental import pallas as pl
from jax.experimental.pallas import tpu as pltpu
from jax.experimental.pallas import tpu_sc as plsc

N_NODES = 10000
N_EDGES = 320000
D = 128

NC = 2
NS = 16
DH = D // NC
CHUNK = 128
NROWS = N_EDGES // CHUNK
STEPS = 156
PH = 12
NPHASE = STEPS // PH
NTAIL = NROWS - NS * STEPS
N_ACC = 10240
ROWS_PER_TILE = N_ACC // NS


def _make_sc_agg(with_deg):
  mesh = plsc.VectorSubcoreMesh(core_axis_name="c", subcore_axis_name="s")
  out_type = jax.ShapeDtypeStruct((NC, N_ACC, DH), jnp.float32)
  if with_deg:
    out_type = (out_type, jax.ShapeDtypeStruct((NS, N_ACC), jnp.float32))
  scratch = [
      pltpu.VMEM((2, PH, CHUNK), jnp.int32),
      pltpu.VMEM((2, PH, CHUNK), jnp.int32),
      pltpu.VMEM((3, CHUNK, DH), jnp.float32),
      pltpu.VMEM_SHARED((N_NODES, DH), jnp.float32),
      pltpu.VMEM_SHARED((N_ACC, DH), jnp.float32),
      pltpu.SemaphoreType.DMA((2,)),
      pltpu.SemaphoreType.DMA,
      pltpu.SemaphoreType.DMA,
  ]
  if with_deg:
    scratch.append(pltpu.VMEM((N_ACC,), jnp.float32))

  @functools.partial(
      pl.kernel, out_type=out_type, mesh=mesh, scratch_types=scratch,
      compiler_params=pltpu.CompilerParams(needs_layout_passes=False,
                                           use_tc_tiling_on_sc=False))
  def agg(table_hbm, edges_hbm, *rest):
    if with_deg:
      (out_hbm, deg_hbm, src_v, dst_v, rows, table, acc,
       gsem, ssem, psem, hist) = rest
    else:
      (out_hbm, src_v, dst_v, rows, table, acc,
       gsem, ssem, psem) = rest
    c = lax.axis_index("c")
    s = lax.axis_index("s")
    my_rows = s * ROWS_PER_TILE

    tstart = jnp.minimum(s * 640, N_NODES - 640)
    pltpu.async_copy(table_hbm.at[pl.ds(tstart, 640), pl.ds(c * DH, DH)],
                     table.at[pl.ds(tstart, 640)], psem)
    base = s * STEPS
    pltpu.async_copy(edges_hbm.at[0, pl.ds(base, PH)], src_v.at[0], psem)
    pltpu.async_copy(edges_hbm.at[1, pl.ds(base, PH)], dst_v.at[0], psem)

    zeros16 = jnp.zeros((16,), jnp.float32)

    def zrows(r, carry):
      for k in range(DH // 16):
        rows[0, r, pl.ds(k * 16, 16)] = zeros16
      return carry

    lax.fori_loop(0, CHUNK, zrows, 0)

    def zloop(i, carry):
      pltpu.sync_copy(rows.at[0], acc.at[pl.ds(my_rows + i * CHUNK, CHUNK)])
      return carry

    lax.fori_loop(0, ROWS_PER_TILE // CHUNK, zloop, 0)

    if with_deg:
      @pl.when(c == 0)
      def _():
        def zhist(i, carry):
          hist[pl.ds(i * 16, 16)] = zeros16
          return carry

        lax.fori_loop(0, N_ACC // 16, zhist, 0)

    pltpu.make_async_copy(table_hbm.at[pl.ds(tstart, 640),
                                       pl.ds(c * DH, DH)],
                          table.at[pl.ds(tstart, 640)], psem).wait()
    pltpu.make_async_copy(edges_hbm.at[0, pl.ds(base, PH)], src_v.at[0],
                          psem).wait()
    pltpu.make_async_copy(edges_hbm.at[1, pl.ds(base, PH)], dst_v.at[0],
                          psem).wait()
    plsc.subcore_barrier()

    ones16 = jnp.ones((16,), jnp.float32)

    def gstart(pb, j, buf):
      pltpu.async_copy(table.at[src_v.at[pb, j]], rows.at[buf],
                       gsem.at[lax.rem(j, 2)])

    def gwait(j):
      pltpu.make_async_copy(table.at[src_v.at[0, 0]], rows.at[0],
                            gsem.at[lax.rem(j, 2)]).wait()

    def swait():
      pltpu.make_async_copy(rows.at[0], acc.at[dst_v.at[0, 0]], ssem).wait()

    def hist_add(pb, j):
      if with_deg:
        @pl.when(c == 0)
        def _():
          for k in range(CHUNK // 16):
            idx = dst_v[pb, j, pl.ds(k * 16, 16)]
            plsc.addupdate_scatter(hist, [idx], ones16)

    def phase_body(p, carry):
      pb = lax.rem(p, 2)
      pn = lax.rem(p + 1, 2)
      @pl.when(p + 1 < NPHASE)
      def _():
        nxt = base + (p + 1) * PH
        pltpu.async_copy(edges_hbm.at[0, pl.ds(nxt, PH)], src_v.at[pn], psem)
        pltpu.async_copy(edges_hbm.at[1, pl.ds(nxt, PH)], dst_v.at[pn], psem)

      gstart(pb, 0, 0)
      gstart(pb, 1, 1)

      def body(j, carry2):
        bj = lax.rem(j, 3)
        @pl.when(j >= 1)
        def _():
          swait()

        gwait(j)
        pltpu.async_copy(rows.at[bj], acc.at[dst_v.at[pb, j]], ssem,
                         add=True)
        hist_add(pb, j)

        @pl.when(j + 2 < PH)
        def _():
          gstart(pb, j + 2, lax.rem(j + 2, 3))

        return carry2

      lax.fori_loop(0, PH, body, 0)
      swait()

      @pl.when(p + 1 < NPHASE)
      def _():
        pltpu.make_async_copy(edges_hbm.at[0, pl.ds(base, PH)], src_v.at[0],
                              psem).wait()
        pltpu.make_async_copy(edges_hbm.at[1, pl.ds(base, PH)], dst_v.at[0],
                              psem).wait()

      return carry

    lax.fori_loop(0, NPHASE, phase_body, 0)

    @pl.when(s < NTAIL)
    def _():
      trow = NS * STEPS + s
      pltpu.sync_copy(edges_hbm.at[0, pl.ds(trow, 1)],
                      src_v.at[0, pl.ds(0, 1)])
      pltpu.sync_copy(edges_hbm.at[1, pl.ds(trow, 1)],
                      dst_v.at[0, pl.ds(0, 1)])
      pltpu.sync_copy(table.at[src_v.at[0, 0]], rows.at[0])
      pltpu.sync_copy(rows.at[0], acc.at[dst_v.at[0, 0]], add=True)
      hist_add(0, 0)

    plsc.subcore_barrier()

    pltpu.sync_copy(acc.at[pl.ds(my_rows, ROWS_PER_TILE)],
                    out_hbm.at[c, pl.ds(my_rows, ROWS_PER_TILE)])
    if with_deg:
      @pl.when(c == 0)
      def _():
        pltpu.sync_copy(hist, deg_hbm.at[s])

  return agg


_sc_agg_l1 = _make_sc_agg(True)
_sc_agg_l2 = _make_sc_agg(False)

NP2 = N_NODES // 2


def _deg_scale(deg_ref):
  deg_cols = jnp.transpose(deg_ref[...])[:N_NODES]
  deg = jnp.sum(deg_cols, axis=1, keepdims=True)
  inv2 = (1.0 / jnp.maximum(deg, 1.0)).reshape(NP2, 2)
  return jnp.concatenate(
      [jnp.broadcast_to(inv2[:, 0:1], (NP2, DH)),
       jnp.broadcast_to(inv2[:, 1:2], (NP2, DH))], axis=1)


def _mean_agg_matmul(aggp_ref, scale, wl_ref):
  zz = jnp.zeros((DH, D), jnp.float32)
  packed = None
  for cc in range(NC):
    wl_c = wl_ref[pl.ds(cc * DH, DH), :]
    wbig = jnp.concatenate(
        [jnp.concatenate([wl_c, zz], axis=1),
         jnp.concatenate([zz, wl_c], axis=1)], axis=0)
    term = jnp.dot(aggp_ref[cc, :NP2, :] * scale, wbig,
                   preferred_element_type=jnp.float32)
    packed = term if packed is None else packed + term
  return packed.reshape(N_NODES, D)


def _tc_layer1(aggp, degp, x, wl, wr, b, gamma, beta):
  def body(aggp_ref, deg_ref, x_ref, wl_ref, wr_ref, b_ref, g_ref, be_ref,
           h_ref, scale_ref):
    scale = _deg_scale(deg_ref)
    scale_ref[...] = scale
    h = (_mean_agg_matmul(aggp_ref, scale, wl_ref)
         + b_ref[...][None, :]
         + jnp.dot(x_ref[...], wr_ref[...],
                   preferred_element_type=jnp.float32))
    mu = jnp.mean(h, axis=0)
    var = jnp.mean((h - mu[None, :]) ** 2, axis=0)
    hn = (h - mu[None, :]) / jnp.sqrt(var + 1e-5)
    hn = hn * g_ref[...][None, :] + be_ref[...][None, :]
    h_ref[...] = jnp.maximum(hn, 0.0)

  return pl.pallas_call(
      body,
      out_shape=(jax.ShapeDtypeStruct((N_NODES, D), jnp.float32),
                 jax.ShapeDtypeStruct((NP2, 2 * DH), jnp.float32)),
  )(aggp, degp, x, wl, wr, b, gamma, beta)


def _tc_layer2(aggp2, scale, h, wl, wr, b):
  def body(aggp2_ref, scale_ref, h_ref, wl_ref, wr_ref, b_ref, o_ref):
    o_ref[...] = (
        _mean_agg_matmul(aggp2_ref, scale_ref[...], wl_ref)
        + b_ref[...][None, :]
        + jnp.dot(h_ref[...], wr_ref[...], preferred_element_type=jnp.float32))

  return pl.pallas_call(
      body,
      out_shape=jax.ShapeDtypeStruct((N_NODES, D), jnp.float32),
  )(aggp2, scale, h, wl, wr, b)


def kernel(x, edge_index, W1l, W1r, b1, gamma1, beta1, W2l, W2r, b2):
  edges = edge_index.astype(jnp.int32).reshape(2, NROWS, CHUNK)

  aggp1, degp = _sc_agg_l1(x, edges)
  aggp1 = aggp1.reshape(NC, N_ACC // 2, 2 * DH)
  h, scale = _tc_layer1(aggp1, degp, x, W1l, W1r, b1, gamma1, beta1)
  aggp2 = _sc_agg_l2(h, edges).reshape(NC, N_ACC // 2, 2 * DH)
  return _tc_layer2(aggp2, scale, h, W2l, W2r, b2)


# speedup vs baseline: 12.6755x; 1.2029x over previous
import functools

import jax
import jax.numpy as jnp
from jax import lax
from jax.experimental import pallas as pl
from jax.experimental.pallas import tpu as pltpu
from jax.experimental.pallas import tpu_sc as plsc

N_NODES = 10000
N_EDGES = 320000
D = 128

NC = 2
NS = 16
DH = D // NC
CHUNK = 128
NROWS = N_EDGES // CHUNK
STEPS = 156
PH = 12
NPHASE = STEPS // PH
NTAIL = NROWS - NS * STEPS
N_ACC = 10240
ROWS_PER_TILE = N_ACC // NS


def _make_sc_agg(with_deg):
  mesh = plsc.VectorSubcoreMesh(core_axis_name="c", subcore_axis_name="s")
  out_type = jax.ShapeDtypeStruct((NC, N_ACC, DH), jnp.float32)
  if with_deg:
    out_type = (out_type, jax.ShapeDtypeStruct((NS, N_ACC), jnp.float32))
  scratch = [
      pltpu.VMEM((2, PH, CHUNK), jnp.int32),
      pltpu.VMEM((2, PH, CHUNK), jnp.int32),
      pltpu.VMEM((3, CHUNK, DH), jnp.float32),
      pltpu.VMEM_SHARED((N_ACC, DH), jnp.float32),
      pltpu.SemaphoreType.DMA((2,)),
      pltpu.SemaphoreType.DMA,
      pltpu.SemaphoreType.DMA,
  ]
  if with_deg:
    scratch.append(pltpu.VMEM((N_ACC,), jnp.float32))

  @functools.partial(
      pl.kernel, out_type=out_type, mesh=mesh, scratch_types=scratch,
      compiler_params=pltpu.CompilerParams(needs_layout_passes=False,
                                           use_tc_tiling_on_sc=False))
  def agg(table_hbm, edges_hbm, *rest):
    if with_deg:
      (out_hbm, deg_hbm, src_v, dst_v, rows, acc,
       gsem, ssem, psem, hist) = rest
    else:
      (out_hbm, src_v, dst_v, rows, acc,
       gsem, ssem, psem) = rest
    c = lax.axis_index("c")
    s = lax.axis_index("s")
    my_rows = s * ROWS_PER_TILE

    base = s * STEPS
    pltpu.async_copy(edges_hbm.at[0, pl.ds(base, PH)], src_v.at[0], psem)
    pltpu.async_copy(edges_hbm.at[1, pl.ds(base, PH)], dst_v.at[0], psem)

    zeros16 = jnp.zeros((16,), jnp.float32)

    def zrows(r, carry):
      for k in range(DH // 16):
        rows[0, r, pl.ds(k * 16, 16)] = zeros16
      return carry

    lax.fori_loop(0, CHUNK, zrows, 0)

    def zloop(i, carry):
      pltpu.sync_copy(rows.at[0], acc.at[pl.ds(my_rows + i * CHUNK, CHUNK)])
      return carry

    lax.fori_loop(0, ROWS_PER_TILE // CHUNK, zloop, 0)

    if with_deg:
      @pl.when(c == 0)
      def _():
        def zhist(i, carry):
          hist[pl.ds(i * 16, 16)] = zeros16
          return carry

        lax.fori_loop(0, N_ACC // 16, zhist, 0)

    pltpu.make_async_copy(edges_hbm.at[0, pl.ds(base, PH)], src_v.at[0],
                          psem).wait()
    pltpu.make_async_copy(edges_hbm.at[1, pl.ds(base, PH)], dst_v.at[0],
                          psem).wait()
    plsc.subcore_barrier()

    ones16 = jnp.ones((16,), jnp.float32)

    def gstart(pb, j, buf):
      pltpu.async_copy(table_hbm.at[c].at[src_v.at[pb, j]],
                       rows.at[buf], gsem.at[lax.rem(j, 2)])

    def gwait(j):
      pltpu.make_async_copy(
          table_hbm.at[c].at[src_v.at[0, 0]], rows.at[0],
          gsem.at[lax.rem(j, 2)]).wait()

    def swait():
      pltpu.make_async_copy(rows.at[0], acc.at[dst_v.at[0, 0]], ssem).wait()

    def hist_add(pb, j):
      if with_deg:
        @pl.when(c == 0)
        def _():
          for k in range(CHUNK // 16):
            idx = dst_v[pb, j, pl.ds(k * 16, 16)]
            plsc.addupdate_scatter(hist, [idx], ones16)

    def phase_body(p, carry):
      pb = lax.rem(p, 2)
      pn = lax.rem(p + 1, 2)
      @pl.when(p + 1 < NPHASE)
      def _():
        nxt = base + (p + 1) * PH
        pltpu.async_copy(edges_hbm.at[0, pl.ds(nxt, PH)], src_v.at[pn], psem)
        pltpu.async_copy(edges_hbm.at[1, pl.ds(nxt, PH)], dst_v.at[pn], psem)

      gstart(pb, 0, 0)
      gstart(pb, 1, 1)

      def body(j, carry2):
        bj = lax.rem(j, 3)
        @pl.when(j >= 1)
        def _():
          swait()

        gwait(j)
        pltpu.async_copy(rows.at[bj], acc.at[dst_v.at[pb, j]], ssem,
                         add=True)
        hist_add(pb, j)

        @pl.when(j + 2 < PH)
        def _():
          gstart(pb, j + 2, lax.rem(j + 2, 3))

        return carry2

      lax.fori_loop(0, PH, body, 0)
      swait()

      @pl.when(p + 1 < NPHASE)
      def _():
        pltpu.make_async_copy(edges_hbm.at[0, pl.ds(base, PH)], src_v.at[0],
                              psem).wait()
        pltpu.make_async_copy(edges_hbm.at[1, pl.ds(base, PH)], dst_v.at[0],
                              psem).wait()

      return carry

    lax.fori_loop(0, NPHASE, phase_body, 0)

    @pl.when(s < NTAIL)
    def _():
      trow = NS * STEPS + s
      pltpu.sync_copy(edges_hbm.at[0, pl.ds(trow, 1)],
                      src_v.at[0, pl.ds(0, 1)])
      pltpu.sync_copy(edges_hbm.at[1, pl.ds(trow, 1)],
                      dst_v.at[0, pl.ds(0, 1)])
      pltpu.sync_copy(table_hbm.at[c].at[src_v.at[0, 0]], rows.at[0])
      pltpu.sync_copy(rows.at[0], acc.at[dst_v.at[0, 0]], add=True)
      hist_add(0, 0)

    plsc.subcore_barrier()

    pltpu.sync_copy(acc.at[pl.ds(my_rows, ROWS_PER_TILE)],
                    out_hbm.at[c, pl.ds(my_rows, ROWS_PER_TILE)])
    if with_deg:
      @pl.when(c == 0)
      def _():
        pltpu.sync_copy(hist, deg_hbm.at[s])

  return agg


_sc_agg_l1 = _make_sc_agg(True)
_sc_agg_l2 = _make_sc_agg(False)

NP2 = N_NODES // 2


def _deg_scale(deg_ref):
  deg_cols = jnp.transpose(deg_ref[...])[:N_NODES]
  deg = jnp.sum(deg_cols, axis=1, keepdims=True)
  inv2 = (1.0 / jnp.maximum(deg, 1.0)).reshape(NP2, 2)
  return jnp.concatenate(
      [jnp.broadcast_to(inv2[:, 0:1], (NP2, DH)),
       jnp.broadcast_to(inv2[:, 1:2], (NP2, DH))], axis=1)


def _mean_agg_matmul(aggp_ref, scale, wl_ref):
  zz = jnp.zeros((DH, D), jnp.float32)
  packed = None
  for cc in range(NC):
    wl_c = wl_ref[pl.ds(cc * DH, DH), :]
    wbig = jnp.concatenate(
        [jnp.concatenate([wl_c, zz], axis=1),
         jnp.concatenate([zz, wl_c], axis=1)], axis=0)
    term = jnp.dot(aggp_ref[cc, :NP2, :] * scale, wbig,
                   preferred_element_type=jnp.float32)
    packed = term if packed is None else packed + term
  return packed.reshape(N_NODES, D)


def _tc_layer1(aggp, degp, x, wl, wr, b, gamma, beta):
  def body(aggp_ref, deg_ref, x_ref, wl_ref, wr_ref, b_ref, g_ref, be_ref,
           h_ref, scale_ref):
    scale = _deg_scale(deg_ref)
    scale_ref[...] = scale
    h = (_mean_agg_matmul(aggp_ref, scale, wl_ref)
         + b_ref[...][None, :]
         + jnp.dot(x_ref[...], wr_ref[...],
                   preferred_element_type=jnp.float32))
    mu = jnp.mean(h, axis=0)
    var = jnp.mean((h - mu[None, :]) ** 2, axis=0)
    hn = (h - mu[None, :]) / jnp.sqrt(var + 1e-5)
    hn = hn * g_ref[...][None, :] + be_ref[...][None, :]
    h_ref[...] = jnp.maximum(hn, 0.0)

  return pl.pallas_call(
      body,
      out_shape=(jax.ShapeDtypeStruct((N_NODES, D), jnp.float32),
                 jax.ShapeDtypeStruct((NP2, 2 * DH), jnp.float32)),
  )(aggp, degp, x, wl, wr, b, gamma, beta)


def _tc_layer2(aggp2, scale, h, wl, wr, b):
  def body(aggp2_ref, scale_ref, h_ref, wl_ref, wr_ref, b_ref, o_ref):
    o_ref[...] = (
        _mean_agg_matmul(aggp2_ref, scale_ref[...], wl_ref)
        + b_ref[...][None, :]
        + jnp.dot(h_ref[...], wr_ref[...], preferred_element_type=jnp.float32))

  return pl.pallas_call(
      body,
      out_shape=jax.ShapeDtypeStruct((N_NODES, D), jnp.float32),
  )(aggp2, scale, h, wl, wr, b)


def kernel(x, edge_index, W1l, W1r, b1, gamma1, beta1, W2l, W2r, b2):
  edges = edge_index.astype(jnp.int32).reshape(2, NROWS, CHUNK)

  xh = jnp.stack([x[:, :DH], x[:, DH:]])
  aggp1, degp = _sc_agg_l1(xh, edges)
  aggp1 = aggp1.reshape(NC, N_ACC // 2, 2 * DH)
  h, scale = _tc_layer1(aggp1, degp, x, W1l, W1r, b1, gamma1, beta1)
  hh = jnp.stack([h[:, :DH], h[:, DH:]])
  aggp2 = _sc_agg_l2(hh, edges).reshape(NC, N_ACC // 2, 2 * DH)
  return _tc_layer2(aggp2, scale, h, W2l, W2r, b2)


# speedup vs baseline: 12.9418x; 1.0210x over previous
import functools

import jax
import jax.numpy as jnp
from jax import lax
from jax.experimental import pallas as pl
from jax.experimental.pallas import tpu as pltpu
from jax.experimental.pallas import tpu_sc as plsc

N_NODES = 10000
N_EDGES = 320000
D = 128

NC = 2
NS = 16
DH = D // NC
CHUNK = 128
NROWS = N_EDGES // CHUNK
STEPS = 156
PH = 12
NPHASE = STEPS // PH
NTAIL = NROWS - NS * STEPS
N_ACC = 10240
ROWS_PER_TILE = N_ACC // NS


def _make_sc_agg(with_deg):
  mesh = plsc.VectorSubcoreMesh(core_axis_name="c", subcore_axis_name="s")
  out_type = jax.ShapeDtypeStruct((NC, N_ACC, DH), jnp.float32)
  if with_deg:
    out_type = (out_type, jax.ShapeDtypeStruct((NS, N_ACC), jnp.float32))
  scratch = [
      pltpu.VMEM((2, PH, CHUNK), jnp.int32),
      pltpu.VMEM((2, PH, CHUNK), jnp.int32),
      pltpu.VMEM((3, CHUNK, DH), jnp.float32),
      pltpu.VMEM_SHARED((N_ACC, DH), jnp.float32),
      pltpu.SemaphoreType.DMA((2,)),
      pltpu.SemaphoreType.DMA,
      pltpu.SemaphoreType.DMA,
  ]
  if with_deg:
    scratch.append(pltpu.VMEM((N_ACC,), jnp.float32))

  @functools.partial(
      pl.kernel, out_type=out_type, mesh=mesh, scratch_types=scratch,
      compiler_params=pltpu.CompilerParams(needs_layout_passes=False,
                                           use_tc_tiling_on_sc=False))
  def agg(table_hbm, edges_hbm, *rest):
    if with_deg:
      (out_hbm, deg_hbm, src_v, dst_v, rows, acc,
       gsem, ssem, psem, hist) = rest
    else:
      (out_hbm, src_v, dst_v, rows, acc,
       gsem, ssem, psem) = rest
    c = lax.axis_index("c")
    s = lax.axis_index("s")
    my_rows = s * ROWS_PER_TILE

    base = s * STEPS
    pltpu.async_copy(edges_hbm.at[0, pl.ds(base, PH)], src_v.at[0], psem)
    pltpu.async_copy(edges_hbm.at[1, pl.ds(base, PH)], dst_v.at[0], psem)

    zeros16 = jnp.zeros((16,), jnp.float32)

    def zrows(r, carry):
      for k in range(DH // 16):
        rows[0, r, pl.ds(k * 16, 16)] = zeros16
      return carry

    lax.fori_loop(0, CHUNK, zrows, 0)

    def zloop(i, carry):
      pltpu.sync_copy(rows.at[0], acc.at[pl.ds(my_rows + i * CHUNK, CHUNK)])
      return carry

    lax.fori_loop(0, ROWS_PER_TILE // CHUNK, zloop, 0)

    if with_deg:
      @pl.when(c == 0)
      def _():
        def zhist(i, carry):
          hist[pl.ds(i * 16, 16)] = zeros16
          return carry

        lax.fori_loop(0, N_ACC // 16, zhist, 0)

    pltpu.make_async_copy(edges_hbm.at[0, pl.ds(base, PH)], src_v.at[0],
                          psem).wait()
    pltpu.make_async_copy(edges_hbm.at[1, pl.ds(base, PH)], dst_v.at[0],
                          psem).wait()
    plsc.subcore_barrier()

    ones16 = jnp.ones((16,), jnp.float32)

    def gstart(pb, j, buf):
      pltpu.async_copy(table_hbm.at[c].at[src_v.at[pb, j]],
                       rows.at[buf], gsem.at[lax.rem(j, 2)])

    def gwait(j):
      pltpu.make_async_copy(
          table_hbm.at[c].at[src_v.at[0, 0]], rows.at[0],
          gsem.at[lax.rem(j, 2)]).wait()

    def swait():
      pltpu.make_async_copy(rows.at[0], acc.at[dst_v.at[0, 0]], ssem).wait()

    def hist_add(pb, j):
      if with_deg:
        @pl.when(c == 0)
        def _():
          for k in range(CHUNK // 16):
            idx = dst_v[pb, j, pl.ds(k * 16, 16)]
            plsc.addupdate_scatter(hist, [idx], ones16)

    def phase_body(p, carry):
      pb = lax.rem(p, 2)
      pn = lax.rem(p + 1, 2)
      @pl.when(p + 1 < NPHASE)
      def _():
        nxt = base + (p + 1) * PH
        pltpu.async_copy(edges_hbm.at[0, pl.ds(nxt, PH)], src_v.at[pn], psem)
        pltpu.async_copy(edges_hbm.at[1, pl.ds(nxt, PH)], dst_v.at[pn], psem)

      gstart(pb, 0, 0)
      gstart(pb, 1, 1)

      def body(j, carry2):
        bj = lax.rem(j, 3)
        @pl.when(j >= 1)
        def _():
          swait()

        gwait(j)
        pltpu.async_copy(rows.at[bj], acc.at[dst_v.at[pb, j]], ssem,
                         add=True)
        hist_add(pb, j)

        @pl.when(j + 2 < PH)
        def _():
          gstart(pb, j + 2, lax.rem(j + 2, 3))

        return carry2

      lax.fori_loop(0, PH, body, 0)
      swait()

      @pl.when(p + 1 < NPHASE)
      def _():
        pltpu.make_async_copy(edges_hbm.at[0, pl.ds(base, PH)], src_v.at[0],
                              psem).wait()
        pltpu.make_async_copy(edges_hbm.at[1, pl.ds(base, PH)], dst_v.at[0],
                              psem).wait()

      return carry

    lax.fori_loop(0, NPHASE, phase_body, 0)

    @pl.when(s < NTAIL)
    def _():
      trow = NS * STEPS + s
      pltpu.sync_copy(edges_hbm.at[0, pl.ds(trow, 1)],
                      src_v.at[0, pl.ds(0, 1)])
      pltpu.sync_copy(edges_hbm.at[1, pl.ds(trow, 1)],
                      dst_v.at[0, pl.ds(0, 1)])
      pltpu.sync_copy(table_hbm.at[c].at[src_v.at[0, 0]], rows.at[0])
      pltpu.sync_copy(rows.at[0], acc.at[dst_v.at[0, 0]], add=True)
      hist_add(0, 0)

    plsc.subcore_barrier()

    pltpu.sync_copy(acc.at[pl.ds(my_rows, ROWS_PER_TILE)],
                    out_hbm.at[c, pl.ds(my_rows, ROWS_PER_TILE)])
    if with_deg:
      @pl.when(c == 0)
      def _():
        pltpu.sync_copy(hist, deg_hbm.at[s])

  return agg


_sc_agg_l1 = _make_sc_agg(True)
_sc_agg_l2 = _make_sc_agg(False)

NP2 = N_NODES // 2


def _deg_scale(deg_ref):
  deg_cols = jnp.transpose(deg_ref[...])[:N_NODES]
  deg = jnp.sum(deg_cols, axis=1, keepdims=True)
  inv2 = (1.0 / jnp.maximum(deg, 1.0)).reshape(NP2, 2)
  return jnp.concatenate(
      [jnp.broadcast_to(inv2[:, 0:1], (NP2, DH)),
       jnp.broadcast_to(inv2[:, 1:2], (NP2, DH))], axis=1)


def _mean_agg_matmul(aggp_ref, scale, wl_ref):
  zz = jnp.zeros((DH, D), jnp.float32)
  packed = None
  for cc in range(NC):
    wl_c = wl_ref[pl.ds(cc * DH, DH), :]
    wbig = jnp.concatenate(
        [jnp.concatenate([wl_c, zz], axis=1),
         jnp.concatenate([zz, wl_c], axis=1)], axis=0)
    term = jnp.dot(aggp_ref[cc, :NP2, :] * scale, wbig,
                   preferred_element_type=jnp.float32)
    packed = term if packed is None else packed + term
  return packed.reshape(N_NODES, D)


def _tc_presplit(x, edges_flat):
  def body(x_ref, e_ref, xh_ref, e2_ref):
    xh_ref[0] = x_ref[:, :DH]
    xh_ref[1] = x_ref[:, DH:]
    e2_ref[...] = e_ref[...].reshape(2, NROWS, CHUNK)

  return pl.pallas_call(
      body,
      out_shape=(jax.ShapeDtypeStruct((NC, N_NODES, DH), jnp.float32),
                 jax.ShapeDtypeStruct((2, NROWS, CHUNK), jnp.int32)),
  )(x, edges_flat)


def _tc_layer1(aggp, degp, x, wl, wr, b, gamma, beta):
  def body(aggp_ref, deg_ref, x_ref, wl_ref, wr_ref, b_ref, g_ref, be_ref,
           h_ref, hh_ref, scale_ref):
    scale = _deg_scale(deg_ref)
    scale_ref[...] = scale
    h = (_mean_agg_matmul(aggp_ref, scale, wl_ref)
         + b_ref[...][None, :]
         + jnp.dot(x_ref[...], wr_ref[...],
                   preferred_element_type=jnp.float32))
    mu = jnp.mean(h, axis=0)
    var = jnp.mean((h - mu[None, :]) ** 2, axis=0)
    hn = (h - mu[None, :]) / jnp.sqrt(var + 1e-5)
    hn = hn * g_ref[...][None, :] + be_ref[...][None, :]
    h = jnp.maximum(hn, 0.0)
    h_ref[...] = h
    hh_ref[0] = h[:, :DH]
    hh_ref[1] = h[:, DH:]

  return pl.pallas_call(
      body,
      out_shape=(jax.ShapeDtypeStruct((N_NODES, D), jnp.float32),
                 jax.ShapeDtypeStruct((NC, N_NODES, DH), jnp.float32),
                 jax.ShapeDtypeStruct((NP2, 2 * DH), jnp.float32)),
  )(aggp, degp, x, wl, wr, b, gamma, beta)


def _tc_layer2(aggp2, scale, h, wl, wr, b):
  def body(aggp2_ref, scale_ref, h_ref, wl_ref, wr_ref, b_ref, o_ref):
    o_ref[...] = (
        _mean_agg_matmul(aggp2_ref, scale_ref[...], wl_ref)
        + b_ref[...][None, :]
        + jnp.dot(h_ref[...], wr_ref[...], preferred_element_type=jnp.float32))

  return pl.pallas_call(
      body,
      out_shape=jax.ShapeDtypeStruct((N_NODES, D), jnp.float32),
  )(aggp2, scale, h, wl, wr, b)


def kernel(x, edge_index, W1l, W1r, b1, gamma1, beta1, W2l, W2r, b2):
  xh, edges = _tc_presplit(x, edge_index.astype(jnp.int32))
  aggp1, degp = _sc_agg_l1(xh, edges)
  aggp1 = aggp1.reshape(NC, N_ACC // 2, 2 * DH)
  h, hh, scale = _tc_layer1(aggp1, degp, x, W1l, W1r, b1, gamma1, beta1)
  aggp2 = _sc_agg_l2(hh, edges).reshape(NC, N_ACC // 2, 2 * DH)
  return _tc_layer2(aggp2, scale, h, W2l, W2r, b2)


# speedup vs baseline: 13.7139x; 1.0597x over previous
import functools

import jax
import jax.numpy as jnp
from jax import lax
from jax.experimental import pallas as pl
from jax.experimental.pallas import tpu as pltpu
from jax.experimental.pallas import tpu_sc as plsc

N_NODES = 10000
N_EDGES = 320000
D = 128

NC = 2
NS = 16
DH = D // NC
CHUNK = 128
NROWS = N_EDGES // CHUNK
STEPS = 156
PH = 12
NPHASE = STEPS // PH
NTAIL = NROWS - NS * STEPS
N_ACC = 10240
ROWS_PER_TILE = N_ACC // NS


def _make_sc_agg(with_deg):
  mesh = plsc.VectorSubcoreMesh(core_axis_name="c", subcore_axis_name="s")
  out_type = jax.ShapeDtypeStruct((NC, N_ACC, DH), jnp.float32)
  if with_deg:
    out_type = (out_type, jax.ShapeDtypeStruct((NS, N_ACC), jnp.float32))
  scratch = [
      pltpu.VMEM((2, PH, CHUNK), jnp.int32),
      pltpu.VMEM((2, PH, CHUNK), jnp.int32),
      pltpu.VMEM((3, CHUNK, DH), jnp.float32),
      pltpu.VMEM_SHARED((N_ACC, DH), jnp.float32),
      pltpu.SemaphoreType.DMA((2,)),
      pltpu.SemaphoreType.DMA,
      pltpu.SemaphoreType.DMA,
  ]
  if with_deg:
    scratch.append(pltpu.VMEM((N_ACC,), jnp.float32))

  @functools.partial(
      pl.kernel, out_type=out_type, mesh=mesh, scratch_types=scratch,
      compiler_params=pltpu.CompilerParams(needs_layout_passes=False,
                                           use_tc_tiling_on_sc=False))
  def agg(table_hbm, edges_hbm, *rest):
    if with_deg:
      (out_hbm, deg_hbm, src_v, dst_v, rows, acc,
       gsem, ssem, psem, hist) = rest
    else:
      (out_hbm, src_v, dst_v, rows, acc,
       gsem, ssem, psem) = rest
    c = lax.axis_index("c")
    s = lax.axis_index("s")
    my_rows = s * ROWS_PER_TILE

    def src_to_halfrows(pb):
      def xform(r, carry):
        for k in range(CHUNK // 16):
          v = src_v[pb, r, pl.ds(k * 16, 16)]
          src_v[pb, r, pl.ds(k * 16, 16)] = v + v + c
        return carry

      lax.fori_loop(0, PH, xform, 0)

    base = s * STEPS
    pltpu.async_copy(edges_hbm.at[0, pl.ds(base, PH)], src_v.at[0], psem)
    pltpu.async_copy(edges_hbm.at[1, pl.ds(base, PH)], dst_v.at[0], psem)

    zeros16 = jnp.zeros((16,), jnp.float32)

    def zrows(r, carry):
      for k in range(DH // 16):
        rows[0, r, pl.ds(k * 16, 16)] = zeros16
      return carry

    lax.fori_loop(0, CHUNK, zrows, 0)

    def zloop(i, carry):
      pltpu.sync_copy(rows.at[0], acc.at[pl.ds(my_rows + i * CHUNK, CHUNK)])
      return carry

    lax.fori_loop(0, ROWS_PER_TILE // CHUNK, zloop, 0)

    if with_deg:
      @pl.when(c == 0)
      def _():
        def zhist(i, carry):
          hist[pl.ds(i * 16, 16)] = zeros16
          return carry

        lax.fori_loop(0, N_ACC // 16, zhist, 0)

    pltpu.make_async_copy(edges_hbm.at[0, pl.ds(base, PH)], src_v.at[0],
                          psem).wait()
    pltpu.make_async_copy(edges_hbm.at[1, pl.ds(base, PH)], dst_v.at[0],
                          psem).wait()
    src_to_halfrows(0)
    plsc.subcore_barrier()

    ones16 = jnp.ones((16,), jnp.float32)

    def gstart(pb, j, buf):
      pltpu.async_copy(table_hbm.at[src_v.at[pb, j]],
                       rows.at[buf], gsem.at[lax.rem(j, 2)])

    def gwait(j):
      pltpu.make_async_copy(
          table_hbm.at[src_v.at[0, 0]], rows.at[0],
          gsem.at[lax.rem(j, 2)]).wait()

    def swait():
      pltpu.make_async_copy(rows.at[0], acc.at[dst_v.at[0, 0]], ssem).wait()

    def hist_add(pb, j):
      if with_deg:
        @pl.when(c == 0)
        def _():
          for k in range(CHUNK // 16):
            idx = dst_v[pb, j, pl.ds(k * 16, 16)]
            plsc.addupdate_scatter(hist, [idx], ones16)

    def phase_body(p, carry):
      pb = lax.rem(p, 2)
      pn = lax.rem(p + 1, 2)
      @pl.when(p + 1 < NPHASE)
      def _():
        nxt = base + (p + 1) * PH
        pltpu.async_copy(edges_hbm.at[0, pl.ds(nxt, PH)], src_v.at[pn], psem)
        pltpu.async_copy(edges_hbm.at[1, pl.ds(nxt, PH)], dst_v.at[pn], psem)

      gstart(pb, 0, 0)
      gstart(pb, 1, 1)

      def body(j, carry2):
        bj = lax.rem(j, 3)
        @pl.when(j >= 1)
        def _():
          swait()

        gwait(j)
        pltpu.async_copy(rows.at[bj], acc.at[dst_v.at[pb, j]], ssem,
                         add=True)
        hist_add(pb, j)

        @pl.when(j + 2 < PH)
        def _():
          gstart(pb, j + 2, lax.rem(j + 2, 3))

        return carry2

      lax.fori_loop(0, PH, body, 0)
      swait()

      @pl.when(p + 1 < NPHASE)
      def _():
        pltpu.make_async_copy(edges_hbm.at[0, pl.ds(base, PH)], src_v.at[0],
                              psem).wait()
        pltpu.make_async_copy(edges_hbm.at[1, pl.ds(base, PH)], dst_v.at[0],
                              psem).wait()
        src_to_halfrows(pn)

      return carry

    lax.fori_loop(0, NPHASE, phase_body, 0)

    @pl.when(s < NTAIL)
    def _():
      trow = NS * STEPS + s
      pltpu.sync_copy(edges_hbm.at[0, pl.ds(trow, 1)],
                      src_v.at[0, pl.ds(0, 1)])
      pltpu.sync_copy(edges_hbm.at[1, pl.ds(trow, 1)],
                      dst_v.at[0, pl.ds(0, 1)])
      for k in range(CHUNK // 16):
        v = src_v[0, 0, pl.ds(k * 16, 16)]
        src_v[0, 0, pl.ds(k * 16, 16)] = v + v + c
      pltpu.sync_copy(table_hbm.at[src_v.at[0, 0]], rows.at[0])
      pltpu.sync_copy(rows.at[0], acc.at[dst_v.at[0, 0]], add=True)
      hist_add(0, 0)

    plsc.subcore_barrier()

    pltpu.sync_copy(acc.at[pl.ds(my_rows, ROWS_PER_TILE)],
                    out_hbm.at[c, pl.ds(my_rows, ROWS_PER_TILE)])
    if with_deg:
      @pl.when(c == 0)
      def _():
        pltpu.sync_copy(hist, deg_hbm.at[s])

  return agg


_sc_agg_l1 = _make_sc_agg(True)
_sc_agg_l2 = _make_sc_agg(False)

NP2 = N_NODES // 2


def _deg_scale(deg_ref):
  deg_cols = jnp.transpose(deg_ref[...])[:N_NODES]
  deg = jnp.sum(deg_cols, axis=1, keepdims=True)
  inv2 = (1.0 / jnp.maximum(deg, 1.0)).reshape(NP2, 2)
  return jnp.concatenate(
      [jnp.broadcast_to(inv2[:, 0:1], (NP2, DH)),
       jnp.broadcast_to(inv2[:, 1:2], (NP2, DH))], axis=1)


def _mean_agg_matmul(aggp_ref, scale, wl_ref):
  zz = jnp.zeros((DH, D), jnp.float32)
  packed = None
  for cc in range(NC):
    wl_c = wl_ref[pl.ds(cc * DH, DH), :]
    wbig = jnp.concatenate(
        [jnp.concatenate([wl_c, zz], axis=1),
         jnp.concatenate([zz, wl_c], axis=1)], axis=0)
    term = jnp.dot(aggp_ref[cc, :NP2, :] * scale, wbig,
                   preferred_element_type=jnp.float32)
    packed = term if packed is None else packed + term
  return packed.reshape(N_NODES, D)


def _tc_layer1(aggp, degp, x, wl, wr, b, gamma, beta):
  def body(aggp_ref, deg_ref, x_ref, wl_ref, wr_ref, b_ref, g_ref, be_ref,
           h_ref, scale_ref):
    scale = _deg_scale(deg_ref)
    scale_ref[...] = scale
    h = (_mean_agg_matmul(aggp_ref, scale, wl_ref)
         + b_ref[...][None, :]
         + jnp.dot(x_ref[...], wr_ref[...],
                   preferred_element_type=jnp.float32))
    mu = jnp.mean(h, axis=0)
    var = jnp.mean((h - mu[None, :]) ** 2, axis=0)
    hn = (h - mu[None, :]) / jnp.sqrt(var + 1e-5)
    hn = hn * g_ref[...][None, :] + be_ref[...][None, :]
    h_ref[...] = jnp.maximum(hn, 0.0)

  return pl.pallas_call(
      body,
      out_shape=(jax.ShapeDtypeStruct((N_NODES, D), jnp.float32),
                 jax.ShapeDtypeStruct((NP2, 2 * DH), jnp.float32)),
  )(aggp, degp, x, wl, wr, b, gamma, beta)


def _tc_layer2(aggp2, scale, h, wl, wr, b):
  def body(aggp2_ref, scale_ref, h_ref, wl_ref, wr_ref, b_ref, o_ref):
    o_ref[...] = (
        _mean_agg_matmul(aggp2_ref, scale_ref[...], wl_ref)
        + b_ref[...][None, :]
        + jnp.dot(h_ref[...], wr_ref[...], preferred_element_type=jnp.float32))

  return pl.pallas_call(
      body,
      out_shape=jax.ShapeDtypeStruct((N_NODES, D), jnp.float32),
  )(aggp2, scale, h, wl, wr, b)


def kernel(x, edge_index, W1l, W1r, b1, gamma1, beta1, W2l, W2r, b2):
  edges = edge_index.astype(jnp.int32).reshape(2, NROWS, CHUNK)
  aggp1, degp = _sc_agg_l1(x.reshape(2 * N_NODES, DH), edges)
  aggp1 = aggp1.reshape(NC, N_ACC // 2, 2 * DH)
  h, scale = _tc_layer1(aggp1, degp, x, W1l, W1r, b1, gamma1, beta1)
  aggp2 = _sc_agg_l2(h.reshape(2 * N_NODES, DH), edges)
  return _tc_layer2(aggp2.reshape(NC, N_ACC // 2, 2 * DH), scale, h,
                    W2l, W2r, b2)


# speedup vs baseline: 13.9196x; 1.0150x over previous
import functools

import jax
import jax.numpy as jnp
from jax import lax
from jax.experimental import pallas as pl
from jax.experimental.pallas import tpu as pltpu
from jax.experimental.pallas import tpu_sc as plsc

N_NODES = 10000
N_EDGES = 320000
D = 128

NC = 2
NS = 16
DH = D // NC
CHUNK = 128
NROWS = N_EDGES // CHUNK
STEPS = 156
PH = 12
NPHASE = STEPS // PH
NTAIL = NROWS - NS * STEPS
N_ACC = 10240
ROWS_PER_TILE = N_ACC // NS


def _make_sc_agg(with_deg):
  mesh = plsc.VectorSubcoreMesh(core_axis_name="c", subcore_axis_name="s")
  out_type = jax.ShapeDtypeStruct((NC, N_ACC, DH), jnp.float32)
  if with_deg:
    out_type = (out_type, jax.ShapeDtypeStruct((NS, N_ACC), jnp.float32))
  scratch = [
      pltpu.VMEM((2, PH, CHUNK), jnp.int32),
      pltpu.VMEM((2, PH, CHUNK), jnp.int32),
      pltpu.VMEM((3, CHUNK, DH), jnp.float32),
      pltpu.VMEM_SHARED((N_ACC, DH), jnp.float32),
      pltpu.SemaphoreType.DMA((2,)),
      pltpu.SemaphoreType.DMA,
      pltpu.SemaphoreType.DMA,
  ]
  if with_deg:
    scratch.append(pltpu.VMEM((N_ACC,), jnp.float32))

  @functools.partial(
      pl.kernel, out_type=out_type, mesh=mesh, scratch_types=scratch,
      compiler_params=pltpu.CompilerParams(needs_layout_passes=False,
                                           use_tc_tiling_on_sc=False))
  def agg(table_hbm, edges_hbm, *rest):
    if with_deg:
      (out_hbm, deg_hbm, src_v, dst_v, rows, acc,
       gsem, ssem, psem, hist) = rest
    else:
      (out_hbm, src_v, dst_v, rows, acc,
       gsem, ssem, psem) = rest
    c = lax.axis_index("c")
    s = lax.axis_index("s")
    my_rows = s * ROWS_PER_TILE

    def src_to_halfrows(pb, r):
      for k in range(CHUNK // 16):
        v = src_v[pb, r, pl.ds(k * 16, 16)]
        src_v[pb, r, pl.ds(k * 16, 16)] = v + v + c

    base = s * STEPS
    pltpu.async_copy(edges_hbm.at[0, pl.ds(base, PH)], src_v.at[0], psem)
    pltpu.async_copy(edges_hbm.at[1, pl.ds(base, PH)], dst_v.at[0], psem)

    zeros16 = jnp.zeros((16,), jnp.float32)

    def zrows(r, carry):
      for k in range(DH // 16):
        rows[0, r, pl.ds(k * 16, 16)] = zeros16
      return carry

    lax.fori_loop(0, CHUNK, zrows, 0)

    def zloop(i, carry):
      pltpu.sync_copy(rows.at[0], acc.at[pl.ds(my_rows + i * CHUNK, CHUNK)])
      return carry

    lax.fori_loop(0, ROWS_PER_TILE // CHUNK, zloop, 0)

    if with_deg:
      @pl.when(c == 0)
      def _():
        def zhist(i, carry):
          hist[pl.ds(i * 16, 16)] = zeros16
          return carry

        lax.fori_loop(0, N_ACC // 16, zhist, 0)

    pltpu.make_async_copy(edges_hbm.at[0, pl.ds(base, PH)], src_v.at[0],
                          psem).wait()
    pltpu.make_async_copy(edges_hbm.at[1, pl.ds(base, PH)], dst_v.at[0],
                          psem).wait()
    plsc.subcore_barrier()

    ones16 = jnp.ones((16,), jnp.float32)

    def gstart(pb, j, buf):
      pltpu.async_copy(table_hbm.at[src_v.at[pb, j]],
                       rows.at[buf], gsem.at[lax.rem(j, 2)])

    def gwait(j):
      pltpu.make_async_copy(
          table_hbm.at[src_v.at[0, 0]], rows.at[0],
          gsem.at[lax.rem(j, 2)]).wait()

    def swait():
      pltpu.make_async_copy(rows.at[0], acc.at[dst_v.at[0, 0]], ssem).wait()

    def hist_add(pb, j):
      if with_deg:
        @pl.when(c == 0)
        def _():
          for k in range(CHUNK // 16):
            idx = dst_v[pb, j, pl.ds(k * 16, 16)]
            plsc.addupdate_scatter(hist, [idx], ones16)

    def phase_body(p, carry):
      pb = lax.rem(p, 2)
      pn = lax.rem(p + 1, 2)
      @pl.when(p + 1 < NPHASE)
      def _():
        nxt = base + (p + 1) * PH
        pltpu.async_copy(edges_hbm.at[0, pl.ds(nxt, PH)], src_v.at[pn], psem)
        pltpu.async_copy(edges_hbm.at[1, pl.ds(nxt, PH)], dst_v.at[pn], psem)

      src_to_halfrows(pb, 0)
      gstart(pb, 0, 0)
      src_to_halfrows(pb, 1)
      gstart(pb, 1, 1)

      def body(j, carry2):
        bj = lax.rem(j, 3)
        @pl.when(j + 2 < PH)
        def _():
          src_to_halfrows(pb, j + 2)

        @pl.when(j >= 1)
        def _():
          swait()

        gwait(j)
        pltpu.async_copy(rows.at[bj], acc.at[dst_v.at[pb, j]], ssem,
                         add=True)
        hist_add(pb, j)

        @pl.when(j + 2 < PH)
        def _():
          gstart(pb, j + 2, lax.rem(j + 2, 3))

        return carry2

      lax.fori_loop(0, PH, body, 0)
      swait()

      @pl.when(p + 1 < NPHASE)
      def _():
        pltpu.make_async_copy(edges_hbm.at[0, pl.ds(base, PH)], src_v.at[0],
                              psem).wait()
        pltpu.make_async_copy(edges_hbm.at[1, pl.ds(base, PH)], dst_v.at[0],
                              psem).wait()

      return carry

    lax.fori_loop(0, NPHASE, phase_body, 0)

    @pl.when(s < NTAIL)
    def _():
      trow = NS * STEPS + s
      pltpu.sync_copy(edges_hbm.at[0, pl.ds(trow, 1)],
                      src_v.at[0, pl.ds(0, 1)])
      pltpu.sync_copy(edges_hbm.at[1, pl.ds(trow, 1)],
                      dst_v.at[0, pl.ds(0, 1)])
      src_to_halfrows(0, 0)
      pltpu.sync_copy(table_hbm.at[src_v.at[0, 0]], rows.at[0])
      pltpu.sync_copy(rows.at[0], acc.at[dst_v.at[0, 0]], add=True)
      hist_add(0, 0)

    plsc.subcore_barrier()

    pltpu.sync_copy(acc.at[pl.ds(my_rows, ROWS_PER_TILE)],
                    out_hbm.at[c, pl.ds(my_rows, ROWS_PER_TILE)])
    if with_deg:
      @pl.when(c == 0)
      def _():
        pltpu.sync_copy(hist, deg_hbm.at[s])

  return agg


_sc_agg_l1 = _make_sc_agg(True)
_sc_agg_l2 = _make_sc_agg(False)

NP2 = N_NODES // 2


def _deg_scale(deg_ref):
  deg = jnp.sum(deg_ref[...], axis=0)[None, :]
  inv = 1.0 / jnp.maximum(deg, 1.0)
  inv_col = jnp.transpose(inv)
  inv2 = inv_col[:N_NODES].reshape(NP2, 2)
  return jnp.concatenate(
      [jnp.broadcast_to(inv2[:, 0:1], (NP2, DH)),
       jnp.broadcast_to(inv2[:, 1:2], (NP2, DH))], axis=1)


def _mean_agg_matmul(aggp_ref, scale, wl_ref):
  zz = jnp.zeros((DH, D), jnp.float32)
  packed = None
  for cc in range(NC):
    wl_c = wl_ref[pl.ds(cc * DH, DH), :]
    wbig = jnp.concatenate(
        [jnp.concatenate([wl_c, zz], axis=1),
         jnp.concatenate([zz, wl_c], axis=1)], axis=0)
    term = jnp.dot(aggp_ref[cc, :NP2, :] * scale, wbig,
                   preferred_element_type=jnp.float32)
    packed = term if packed is None else packed + term
  return packed.reshape(N_NODES, D)


def _tc_layer1(aggp, degp, x, wl, wr, b, gamma, beta):
  def body(aggp_ref, deg_ref, x_ref, wl_ref, wr_ref, b_ref, g_ref, be_ref,
           h_ref, scale_ref):
    scale = _deg_scale(deg_ref)
    scale_ref[...] = scale
    h = (_mean_agg_matmul(aggp_ref, scale, wl_ref)
         + b_ref[...][None, :]
         + jnp.dot(x_ref[...], wr_ref[...],
                   preferred_element_type=jnp.float32))
    mu = jnp.mean(h, axis=0)
    var = jnp.mean((h - mu[None, :]) ** 2, axis=0)
    hn = (h - mu[None, :]) / jnp.sqrt(var + 1e-5)
    hn = hn * g_ref[...][None, :] + be_ref[...][None, :]
    h_ref[...] = jnp.maximum(hn, 0.0)

  return pl.pallas_call(
      body,
      out_shape=(jax.ShapeDtypeStruct((N_NODES, D), jnp.float32),
                 jax.ShapeDtypeStruct((NP2, 2 * DH), jnp.float32)),
  )(aggp, degp, x, wl, wr, b, gamma, beta)


def _tc_layer2(aggp2, scale, h, wl, wr, b):
  def body(aggp2_ref, scale_ref, h_ref, wl_ref, wr_ref, b_ref, o_ref):
    o_ref[...] = (
        _mean_agg_matmul(aggp2_ref, scale_ref[...], wl_ref)
        + b_ref[...][None, :]
        + jnp.dot(h_ref[...], wr_ref[...], preferred_element_type=jnp.float32))

  return pl.pallas_call(
      body,
      out_shape=jax.ShapeDtypeStruct((N_NODES, D), jnp.float32),
  )(aggp2, scale, h, wl, wr, b)


def kernel(x, edge_index, W1l, W1r, b1, gamma1, beta1, W2l, W2r, b2):
  edges = edge_index.astype(jnp.int32).reshape(2, NROWS, CHUNK)
  aggp1, degp = _sc_agg_l1(x.reshape(2 * N_NODES, DH), edges)
  aggp1 = aggp1.reshape(NC, N_ACC // 2, 2 * DH)
  h, scale = _tc_layer1(aggp1, degp, x, W1l, W1r, b1, gamma1, beta1)
  aggp2 = _sc_agg_l2(h.reshape(2 * N_NODES, DH), edges)
  return _tc_layer2(aggp2.reshape(NC, N_ACC // 2, 2 * DH), scale, h,
                    W2l, W2r, b2)


# speedup vs baseline: 14.9453x; 1.0737x over previous
import functools

import jax
import jax.numpy as jnp
from jax import lax
from jax.experimental import pallas as pl
from jax.experimental.pallas import tpu as pltpu
from jax.experimental.pallas import tpu_sc as plsc

N_NODES = 10000
N_EDGES = 320000
D = 128

NC = 2
NS = 16
DH = D // NC
CHUNK = 128
NROWS = N_EDGES // CHUNK
STEPS = 156
PH = 52
NPHASE = STEPS // PH
NTAIL = NROWS - NS * STEPS
N_ACC = 10240
ROWS_PER_TILE = N_ACC // NS


def _make_sc_agg(with_deg):
  mesh = plsc.VectorSubcoreMesh(core_axis_name="c", subcore_axis_name="s")
  out_type = jax.ShapeDtypeStruct((NC, N_ACC, DH), jnp.float32)
  if with_deg:
    out_type = (out_type, jax.ShapeDtypeStruct((NS, N_ACC), jnp.float32))
  scratch = [
      pltpu.VMEM((2, PH, CHUNK), jnp.int32),
      pltpu.VMEM((2, PH, CHUNK), jnp.int32),
      pltpu.VMEM((3, CHUNK, DH), jnp.float32),
      pltpu.VMEM_SHARED((N_ACC, DH), jnp.float32),
      pltpu.SemaphoreType.DMA((2,)),
      pltpu.SemaphoreType.DMA,
      pltpu.SemaphoreType.DMA,
  ]
  if with_deg:
    scratch.append(pltpu.VMEM((N_ACC,), jnp.float32))

  @functools.partial(
      pl.kernel, out_type=out_type, mesh=mesh, scratch_types=scratch,
      compiler_params=pltpu.CompilerParams(needs_layout_passes=False,
                                           use_tc_tiling_on_sc=False))
  def agg(table_hbm, edges_hbm, *rest):
    if with_deg:
      (out_hbm, deg_hbm, src_v, dst_v, rows, acc,
       gsem, ssem, psem, hist) = rest
    else:
      (out_hbm, src_v, dst_v, rows, acc,
       gsem, ssem, psem) = rest
    c = lax.axis_index("c")
    s = lax.axis_index("s")
    my_rows = s * ROWS_PER_TILE

    def src_to_halfrows(pb, r):
      for k in range(CHUNK // 16):
        v = src_v[pb, r, pl.ds(k * 16, 16)]
        src_v[pb, r, pl.ds(k * 16, 16)] = v + v + c

    base = s * STEPS
    pltpu.async_copy(edges_hbm.at[0, pl.ds(base, PH)], src_v.at[0], psem)
    pltpu.async_copy(edges_hbm.at[1, pl.ds(base, PH)], dst_v.at[0], psem)

    zeros16 = jnp.zeros((16,), jnp.float32)

    def zrows(r, carry):
      for k in range(DH // 16):
        rows[0, r, pl.ds(k * 16, 16)] = zeros16
      return carry

    lax.fori_loop(0, CHUNK, zrows, 0)

    def zloop(i, carry):
      pltpu.sync_copy(rows.at[0], acc.at[pl.ds(my_rows + i * CHUNK, CHUNK)])
      return carry

    lax.fori_loop(0, ROWS_PER_TILE // CHUNK, zloop, 0)

    if with_deg:
      @pl.when(c == 0)
      def _():
        def zhist(i, carry):
          hist[pl.ds(i * 16, 16)] = zeros16
          return carry

        lax.fori_loop(0, N_ACC // 16, zhist, 0)

    pltpu.make_async_copy(edges_hbm.at[0, pl.ds(base, PH)], src_v.at[0],
                          psem).wait()
    pltpu.make_async_copy(edges_hbm.at[1, pl.ds(base, PH)], dst_v.at[0],
                          psem).wait()
    plsc.subcore_barrier()

    ones16 = jnp.ones((16,), jnp.float32)

    def gstart(pb, j, buf):
      pltpu.async_copy(table_hbm.at[src_v.at[pb, j]],
                       rows.at[buf], gsem.at[lax.rem(j, 2)])

    def gwait(j):
      pltpu.make_async_copy(
          table_hbm.at[src_v.at[0, 0]], rows.at[0],
          gsem.at[lax.rem(j, 2)]).wait()

    def swait():
      pltpu.make_async_copy(rows.at[0], acc.at[dst_v.at[0, 0]], ssem).wait()

    def hist_add(pb, j):
      if with_deg:
        @pl.when(c == 0)
        def _():
          for k in range(CHUNK // 16):
            idx = dst_v[pb, j, pl.ds(k * 16, 16)]
            plsc.addupdate_scatter(hist, [idx], ones16)

    def phase_body(p, carry):
      pb = lax.rem(p, 2)
      pn = lax.rem(p + 1, 2)
      @pl.when(p + 1 < NPHASE)
      def _():
        nxt = base + (p + 1) * PH
        pltpu.async_copy(edges_hbm.at[0, pl.ds(nxt, PH)], src_v.at[pn], psem)
        pltpu.async_copy(edges_hbm.at[1, pl.ds(nxt, PH)], dst_v.at[pn], psem)

      src_to_halfrows(pb, 0)
      gstart(pb, 0, 0)
      src_to_halfrows(pb, 1)
      gstart(pb, 1, 1)

      def body(j, carry2):
        bj = lax.rem(j, 3)
        @pl.when(j + 2 < PH)
        def _():
          src_to_halfrows(pb, j + 2)

        @pl.when(j >= 1)
        def _():
          swait()

        gwait(j)
        pltpu.async_copy(rows.at[bj], acc.at[dst_v.at[pb, j]], ssem,
                         add=True)
        hist_add(pb, j)

        @pl.when(j + 2 < PH)
        def _():
          gstart(pb, j + 2, lax.rem(j + 2, 3))

        return carry2

      lax.fori_loop(0, PH, body, 0)
      swait()

      @pl.when(p + 1 < NPHASE)
      def _():
        pltpu.make_async_copy(edges_hbm.at[0, pl.ds(base, PH)], src_v.at[0],
                              psem).wait()
        pltpu.make_async_copy(edges_hbm.at[1, pl.ds(base, PH)], dst_v.at[0],
                              psem).wait()

      return carry

    lax.fori_loop(0, NPHASE, phase_body, 0)

    @pl.when(s < NTAIL)
    def _():
      trow = NS * STEPS + s
      pltpu.sync_copy(edges_hbm.at[0, pl.ds(trow, 1)],
                      src_v.at[0, pl.ds(0, 1)])
      pltpu.sync_copy(edges_hbm.at[1, pl.ds(trow, 1)],
                      dst_v.at[0, pl.ds(0, 1)])
      src_to_halfrows(0, 0)
      pltpu.sync_copy(table_hbm.at[src_v.at[0, 0]], rows.at[0])
      pltpu.sync_copy(rows.at[0], acc.at[dst_v.at[0, 0]], add=True)
      hist_add(0, 0)

    plsc.subcore_barrier()

    pltpu.sync_copy(acc.at[pl.ds(my_rows, ROWS_PER_TILE)],
                    out_hbm.at[c, pl.ds(my_rows, ROWS_PER_TILE)])
    if with_deg:
      @pl.when(c == 0)
      def _():
        pltpu.sync_copy(hist, deg_hbm.at[s])

  return agg


_sc_agg_l1 = _make_sc_agg(True)
_sc_agg_l2 = _make_sc_agg(False)

NP2 = N_NODES // 2


def _deg_scale(deg_ref):
  deg = jnp.sum(deg_ref[...], axis=0)[None, :]
  inv = 1.0 / jnp.maximum(deg, 1.0)
  inv_col = jnp.transpose(inv)
  inv2 = inv_col[:N_NODES].reshape(NP2, 2)
  return jnp.concatenate(
      [jnp.broadcast_to(inv2[:, 0:1], (NP2, DH)),
       jnp.broadcast_to(inv2[:, 1:2], (NP2, DH))], axis=1)


def _mean_agg_matmul(aggp_ref, scale, wl_ref):
  zz = jnp.zeros((DH, D), jnp.float32)
  packed = None
  for cc in range(NC):
    wl_c = wl_ref[pl.ds(cc * DH, DH), :]
    wbig = jnp.concatenate(
        [jnp.concatenate([wl_c, zz], axis=1),
         jnp.concatenate([zz, wl_c], axis=1)], axis=0)
    term = jnp.dot(aggp_ref[cc, :NP2, :] * scale, wbig,
                   preferred_element_type=jnp.float32)
    packed = term if packed is None else packed + term
  return packed.reshape(N_NODES, D)


def _tc_layer1(aggp, degp, x, wl, wr, b, gamma, beta):
  def body(aggp_ref, deg_ref, x_ref, wl_ref, wr_ref, b_ref, g_ref, be_ref,
           h_ref, scale_ref):
    scale = _deg_scale(deg_ref)
    scale_ref[...] = scale
    h = (_mean_agg_matmul(aggp_ref, scale, wl_ref)
         + b_ref[...][None, :]
         + jnp.dot(x_ref[...], wr_ref[...],
                   preferred_element_type=jnp.float32))
    mu = jnp.mean(h, axis=0)
    var = jnp.mean((h - mu[None, :]) ** 2, axis=0)
    hn = (h - mu[None, :]) / jnp.sqrt(var + 1e-5)
    hn = hn * g_ref[...][None, :] + be_ref[...][None, :]
    h_ref[...] = jnp.maximum(hn, 0.0)

  return pl.pallas_call(
      body,
      out_shape=(jax.ShapeDtypeStruct((N_NODES, D), jnp.float32),
                 jax.ShapeDtypeStruct((NP2, 2 * DH), jnp.float32)),
  )(aggp, degp, x, wl, wr, b, gamma, beta)


def _tc_layer2(aggp2, scale, h, wl, wr, b):
  def body(aggp2_ref, scale_ref, h_ref, wl_ref, wr_ref, b_ref, o_ref):
    o_ref[...] = (
        _mean_agg_matmul(aggp2_ref, scale_ref[...], wl_ref)
        + b_ref[...][None, :]
        + jnp.dot(h_ref[...], wr_ref[...], preferred_element_type=jnp.float32))

  return pl.pallas_call(
      body,
      out_shape=jax.ShapeDtypeStruct((N_NODES, D), jnp.float32),
  )(aggp2, scale, h, wl, wr, b)


def kernel(x, edge_index, W1l, W1r, b1, gamma1, beta1, W2l, W2r, b2):
  edges = edge_index.astype(jnp.int32).reshape(2, NROWS, CHUNK)
  aggp1, degp = _sc_agg_l1(x.reshape(2 * N_NODES, DH), edges)
  aggp1 = aggp1.reshape(NC, N_ACC // 2, 2 * DH)
  h, scale = _tc_layer1(aggp1, degp, x, W1l, W1r, b1, gamma1, beta1)
  aggp2 = _sc_agg_l2(h.reshape(2 * N_NODES, DH), edges)
  return _tc_layer2(aggp2.reshape(NC, N_ACC // 2, 2 * DH), scale, h,
                    W2l, W2r, b2)


# speedup vs baseline: 15.1137x; 1.0113x over previous
import functools

import jax
import jax.numpy as jnp
from jax import lax
from jax.experimental import pallas as pl
from jax.experimental.pallas import tpu as pltpu
from jax.experimental.pallas import tpu_sc as plsc

N_NODES = 10000
N_EDGES = 320000
D = 128

NC = 2
NS = 16
DH = D // NC
CHUNK = 128
NROWS = N_EDGES // CHUNK
STEPS = 156
PH = 78
NPHASE = STEPS // PH
NTAIL = NROWS - NS * STEPS
N_ACC = 10240
ROWS_PER_TILE = N_ACC // NS


def _make_sc_agg(with_deg):
  mesh = plsc.VectorSubcoreMesh(core_axis_name="c", subcore_axis_name="s")
  out_type = jax.ShapeDtypeStruct((NC, N_ACC, DH), jnp.float32)
  if with_deg:
    out_type = (out_type, jax.ShapeDtypeStruct((NS, N_ACC), jnp.float32))
  scratch = [
      pltpu.VMEM((2, PH, CHUNK), jnp.int32),
      pltpu.VMEM((2, PH, CHUNK), jnp.int32),
      pltpu.VMEM((3, CHUNK, DH), jnp.float32),
      pltpu.VMEM_SHARED((N_ACC, DH), jnp.float32),
      pltpu.SemaphoreType.DMA((2,)),
      pltpu.SemaphoreType.DMA,
      pltpu.SemaphoreType.DMA,
  ]
  if with_deg:
    scratch.append(pltpu.VMEM((N_ACC,), jnp.float32))

  @functools.partial(
      pl.kernel, out_type=out_type, mesh=mesh, scratch_types=scratch,
      compiler_params=pltpu.CompilerParams(needs_layout_passes=False,
                                           use_tc_tiling_on_sc=False))
  def agg(table_hbm, edges_hbm, *rest):
    if with_deg:
      (out_hbm, deg_hbm, src_v, dst_v, rows, acc,
       gsem, ssem, psem, hist) = rest
    else:
      (out_hbm, src_v, dst_v, rows, acc,
       gsem, ssem, psem) = rest
    c = lax.axis_index("c")
    s = lax.axis_index("s")
    my_rows = s * ROWS_PER_TILE

    def src_to_halfrows(pb, r):
      for k in range(CHUNK // 16):
        v = src_v[pb, r, pl.ds(k * 16, 16)]
        src_v[pb, r, pl.ds(k * 16, 16)] = v + v + c

    base = s * STEPS
    pltpu.async_copy(edges_hbm.at[0, pl.ds(base, PH)], src_v.at[0], psem)
    pltpu.async_copy(edges_hbm.at[1, pl.ds(base, PH)], dst_v.at[0], psem)

    zeros16 = jnp.zeros((16,), jnp.float32)

    def zrows(r, carry):
      for k in range(DH // 16):
        rows[0, r, pl.ds(k * 16, 16)] = zeros16
      return carry

    lax.fori_loop(0, CHUNK, zrows, 0)

    def zloop(i, carry):
      pltpu.sync_copy(rows.at[0], acc.at[pl.ds(my_rows + i * CHUNK, CHUNK)])
      return carry

    lax.fori_loop(0, ROWS_PER_TILE // CHUNK, zloop, 0)

    if with_deg:
      @pl.when(c == 0)
      def _():
        def zhist(i, carry):
          hist[pl.ds(i * 16, 16)] = zeros16
          return carry

        lax.fori_loop(0, N_ACC // 16, zhist, 0)

    pltpu.make_async_copy(edges_hbm.at[0, pl.ds(base, PH)], src_v.at[0],
                          psem).wait()
    pltpu.make_async_copy(edges_hbm.at[1, pl.ds(base, PH)], dst_v.at[0],
                          psem).wait()
    plsc.subcore_barrier()

    ones16 = jnp.ones((16,), jnp.float32)

    def gstart(pb, j, buf):
      pltpu.async_copy(table_hbm.at[src_v.at[pb, j]],
                       rows.at[buf], gsem.at[lax.rem(j, 2)])

    def gwait(j):
      pltpu.make_async_copy(
          table_hbm.at[src_v.at[0, 0]], rows.at[0],
          gsem.at[lax.rem(j, 2)]).wait()

    def swait():
      pltpu.make_async_copy(rows.at[0], acc.at[dst_v.at[0, 0]], ssem).wait()

    def hist_add(pb, j):
      if with_deg:
        @pl.when(c == 0)
        def _():
          for k in range(CHUNK // 16):
            idx = dst_v[pb, j, pl.ds(k * 16, 16)]
            plsc.addupdate_scatter(hist, [idx], ones16)

    def phase_body(p, carry):
      pb = lax.rem(p, 2)
      pn = lax.rem(p + 1, 2)
      @pl.when(p + 1 < NPHASE)
      def _():
        nxt = base + (p + 1) * PH
        pltpu.async_copy(edges_hbm.at[0, pl.ds(nxt, PH)], src_v.at[pn], psem)
        pltpu.async_copy(edges_hbm.at[1, pl.ds(nxt, PH)], dst_v.at[pn], psem)

      src_to_halfrows(pb, 0)
      gstart(pb, 0, 0)
      src_to_halfrows(pb, 1)
      gstart(pb, 1, 1)

      def body(j, carry2):
        bj = lax.rem(j, 3)
        @pl.when(j + 2 < PH)
        def _():
          src_to_halfrows(pb, j + 2)

        @pl.when(j >= 1)
        def _():
          swait()

        gwait(j)
        pltpu.async_copy(rows.at[bj], acc.at[dst_v.at[pb, j]], ssem,
                         add=True)
        hist_add(pb, j)

        @pl.when(j + 2 < PH)
        def _():
          gstart(pb, j + 2, lax.rem(j + 2, 3))

        return carry2

      lax.fori_loop(0, PH, body, 0)
      swait()

      @pl.when(p + 1 < NPHASE)
      def _():
        pltpu.make_async_copy(edges_hbm.at[0, pl.ds(base, PH)], src_v.at[0],
                              psem).wait()
        pltpu.make_async_copy(edges_hbm.at[1, pl.ds(base, PH)], dst_v.at[0],
                              psem).wait()

      return carry

    lax.fori_loop(0, NPHASE, phase_body, 0)

    @pl.when(s < NTAIL)
    def _():
      trow = NS * STEPS + s
      pltpu.sync_copy(edges_hbm.at[0, pl.ds(trow, 1)],
                      src_v.at[0, pl.ds(0, 1)])
      pltpu.sync_copy(edges_hbm.at[1, pl.ds(trow, 1)],
                      dst_v.at[0, pl.ds(0, 1)])
      src_to_halfrows(0, 0)
      pltpu.sync_copy(table_hbm.at[src_v.at[0, 0]], rows.at[0])
      pltpu.sync_copy(rows.at[0], acc.at[dst_v.at[0, 0]], add=True)
      hist_add(0, 0)

    plsc.subcore_barrier()

    pltpu.sync_copy(acc.at[pl.ds(my_rows, ROWS_PER_TILE)],
                    out_hbm.at[c, pl.ds(my_rows, ROWS_PER_TILE)])
    if with_deg:
      @pl.when(c == 0)
      def _():
        pltpu.sync_copy(hist, deg_hbm.at[s])

  return agg


_sc_agg_l1 = _make_sc_agg(True)
_sc_agg_l2 = _make_sc_agg(False)

NP2 = N_NODES // 2


def _deg_scale(deg_ref):
  deg = jnp.sum(deg_ref[...], axis=0)[None, :]
  inv = 1.0 / jnp.maximum(deg, 1.0)
  inv_col = jnp.transpose(inv)
  inv2 = inv_col[:N_NODES].reshape(NP2, 2)
  return jnp.concatenate(
      [jnp.broadcast_to(inv2[:, 0:1], (NP2, DH)),
       jnp.broadcast_to(inv2[:, 1:2], (NP2, DH))], axis=1)


def _mean_agg_matmul(aggp_ref, scale, wl_ref):
  zz = jnp.zeros((DH, D), jnp.float32)
  packed = None
  for cc in range(NC):
    wl_c = wl_ref[pl.ds(cc * DH, DH), :]
    wbig = jnp.concatenate(
        [jnp.concatenate([wl_c, zz], axis=1),
         jnp.concatenate([zz, wl_c], axis=1)], axis=0)
    term = jnp.dot(aggp_ref[cc, :NP2, :] * scale, wbig,
                   preferred_element_type=jnp.float32)
    packed = term if packed is None else packed + term
  return packed.reshape(N_NODES, D)


def _tc_layer1(aggp, degp, x, wl, wr, b, gamma, beta):
  def body(aggp_ref, deg_ref, x_ref, wl_ref, wr_ref, b_ref, g_ref, be_ref,
           h_ref, scale_ref):
    scale = _deg_scale(deg_ref)
    scale_ref[...] = scale
    h = (_mean_agg_matmul(aggp_ref, scale, wl_ref)
         + b_ref[...][None, :]
         + jnp.dot(x_ref[...], wr_ref[...],
                   preferred_element_type=jnp.float32))
    mu = jnp.mean(h, axis=0)
    var = jnp.mean((h - mu[None, :]) ** 2, axis=0)
    hn = (h - mu[None, :]) / jnp.sqrt(var + 1e-5)
    hn = hn * g_ref[...][None, :] + be_ref[...][None, :]
    h_ref[...] = jnp.maximum(hn, 0.0)

  return pl.pallas_call(
      body,
      out_shape=(jax.ShapeDtypeStruct((N_NODES, D), jnp.float32),
                 jax.ShapeDtypeStruct((NP2, 2 * DH), jnp.float32)),
  )(aggp, degp, x, wl, wr, b, gamma, beta)


def _tc_layer2(aggp2, scale, h, wl, wr, b):
  def body(aggp2_ref, scale_ref, h_ref, wl_ref, wr_ref, b_ref, o_ref):
    o_ref[...] = (
        _mean_agg_matmul(aggp2_ref, scale_ref[...], wl_ref)
        + b_ref[...][None, :]
        + jnp.dot(h_ref[...], wr_ref[...], preferred_element_type=jnp.float32))

  return pl.pallas_call(
      body,
      out_shape=jax.ShapeDtypeStruct((N_NODES, D), jnp.float32),
  )(aggp2, scale, h, wl, wr, b)


def kernel(x, edge_index, W1l, W1r, b1, gamma1, beta1, W2l, W2r, b2):
  edges = edge_index.astype(jnp.int32).reshape(2, NROWS, CHUNK)
  aggp1, degp = _sc_agg_l1(x.reshape(2 * N_NODES, DH), edges)
  aggp1 = aggp1.reshape(NC, N_ACC // 2, 2 * DH)
  h, scale = _tc_layer1(aggp1, degp, x, W1l, W1r, b1, gamma1, beta1)
  aggp2 = _sc_agg_l2(h.reshape(2 * N_NODES, DH), edges)
  return _tc_layer2(aggp2.reshape(NC, N_ACC // 2, 2 * DH), scale, h,
                    W2l, W2r, b2)
